# baseline (speedup 1.0000x reference)
import functools

import jax
import jax.numpy as jnp
from jax import lax
from jax.experimental import pallas as pl
from jax.experimental.pallas import tpu as pltpu
from jax.experimental.pallas import tpu_sc as plsc

E = 8
TOPK = 2
T = 2048
D = 1024
FF = 3584

BT = 256
NBMAX = 16 + E - 1
P = NBMAX * BT
FFB = 512
NF = FF // FFB

NTILES = 32
TPT = T // NTILES
LANES = 128


def _routing_body(x_ref, gw_ref, pos0_ref, pos1_ref, w0_ref, w1_ref, meta_ref):
    x = x_ref[...]
    gw = gw_ref[...]
    logits = lax.dot_general(x, gw, (((1,), (1,)), ((), ())),
                             preferred_element_type=jnp.float32)
    lane = lax.broadcasted_iota(jnp.int32, (T, LANES), 1)
    neg = jnp.float32(-1e30)
    logits = jnp.where(lane < E, logits, neg)

    m0 = jnp.max(logits, axis=1, keepdims=True)
    i0 = jnp.min(jnp.where(logits == m0, lane, LANES), axis=1, keepdims=True)
    l2 = jnp.where(lane == i0, neg, logits)
    m1 = jnp.max(l2, axis=1, keepdims=True)
    i1 = jnp.min(jnp.where(l2 == m1, lane, LANES), axis=1, keepdims=True)

    ex = jnp.exp(m1 - m0)
    w0 = 1.0 / (1.0 + ex)
    w1 = ex / (1.0 + ex)

    oh0 = (lane == i0).astype(jnp.float32)
    oh1 = (lane == i1).astype(jnp.float32)
    cnt = oh0 + oh1

    csum = cnt
    s = 1
    while s < T:
        csum = csum + jnp.concatenate(
            [jnp.zeros((s, LANES), jnp.float32), csum[:-s, :]], axis=0)
        s *= 2
    excl = csum - cnt
    counts = csum[T - 1:T, :]

    nb = jnp.floor((counts + (BT - 1)) / BT)
    nb = jnp.where(lane[:1, :] < E, nb, 0.0)
    pnb = nb
    s = 1
    while s < LANES:
        pnb = pnb + jnp.concatenate(
            [jnp.zeros((1, s), jnp.float32), pnb[:, :-s]], axis=1)
        s *= 2
    pext = pnb - nb
    padded_off = BT * pext

    slot = excl + padded_off
    pos0 = jnp.sum(oh0 * slot, axis=1, keepdims=True)
    pos1 = jnp.sum(oh1 * slot, axis=1, keepdims=True)

    nact = jnp.sum(jnp.where(lane[:1, :] == E - 1, pnb, 0.0),
                   axis=1, keepdims=True)
    bf = lane[:1, :].astype(jnp.float32)
    be = jnp.zeros((1, LANES), jnp.float32)
    for e in range(E):
        pnb_e = jnp.sum(jnp.where(lane[:1, :] == e, pnb, 0.0),
                        axis=1, keepdims=True)
        be = be + (bf >= pnb_e).astype(jnp.float32)
    be = jnp.minimum(be, float(E - 1))

    meta = jnp.where(lane[:1, :] < NBMAX, be,
                     jnp.where(lane[:1, :] == NBMAX, nact, 0.0))

    pos0_ref[...] = pos0.astype(jnp.int32)
    pos1_ref[...] = pos1.astype(jnp.int32)
    w0_ref[...] = jnp.broadcast_to(w0, (T, 16))
    w1_ref[...] = jnp.broadcast_to(w1, (T, 16))
    meta_ref[...] = meta.astype(jnp.int32)


def _routing(x, gw_pad):
    return pl.pallas_call(
        _routing_body,
        out_shape=[
            jax.ShapeDtypeStruct((T, 1), jnp.int32),
            jax.ShapeDtypeStruct((T, 1), jnp.int32),
            jax.ShapeDtypeStruct((T, 16), jnp.float32),
            jax.ShapeDtypeStruct((T, 16), jnp.float32),
            jax.ShapeDtypeStruct((1, LANES), jnp.int32),
        ],
    )(x, gw_pad)


def _dispatch_body(x_hbm, p0_hbm, p1_hbm, out_hbm, idx0_v, idx1_v, rows_v, sem):
    c = lax.axis_index("c")
    s = lax.axis_index("s")
    wid = s * 2 + c
    pltpu.sync_copy(p0_hbm.at[wid], idx0_v)
    pltpu.sync_copy(p1_hbm.at[wid], idx1_v)
    pltpu.sync_copy(x_hbm.at[pl.ds(wid * TPT, TPT)], rows_v)
    pltpu.async_copy(rows_v, out_hbm.at[idx0_v], sem).wait()
    pltpu.async_copy(rows_v, out_hbm.at[idx1_v], sem).wait()


def _dispatch(x, p0, p1):
    mesh = plsc.VectorSubcoreMesh(core_axis_name="c", subcore_axis_name="s")
    fn = functools.partial(
        pl.kernel,
        out_type=jax.ShapeDtypeStruct((P, D), jnp.float32),
        mesh=mesh,
        scratch_types=[
            pltpu.VMEM((TPT,), jnp.int32),
            pltpu.VMEM((TPT,), jnp.int32),
            pltpu.VMEM((TPT, D), jnp.float32),
            pltpu.SemaphoreType.DMA,
        ],
    )(_dispatch_body)
    return fn(x, p0, p1)


def _gmm_body(bex_ref, nact_ref, xs_ref, w1_ref, w3_ref, w2_ref, out_ref):
    b = pl.program_id(0)
    f = pl.program_id(1)

    @pl.when(b < nact_ref[0])
    def _():
        x = xs_ref[...].astype(jnp.bfloat16)
        h1 = lax.dot_general(x, w1_ref[0], (((1,), (1,)), ((), ())),
                             preferred_element_type=jnp.float32)
        h3 = lax.dot_general(x, w3_ref[0], (((1,), (1,)), ((), ())),
                             preferred_element_type=jnp.float32)
        h = (h1 * lax.logistic(h1) * h3).astype(jnp.bfloat16)
        y = lax.dot_general(h, w2_ref[0], (((1,), (1,)), ((), ())),
                            preferred_element_type=jnp.float32)

        @pl.when(f == 0)
        def _():
            out_ref[...] = y

        @pl.when(f > 0)
        def _():
            out_ref[...] += y


def _gmm(bex, nact, xs, w1, w3, w2):
    def expert_of(b, bex_ref, nact_ref):
        return bex_ref[jnp.minimum(b, nact_ref[0] - 1)]

    grid_spec = pltpu.PrefetchScalarGridSpec(
        num_scalar_prefetch=2,
        grid=(NBMAX, NF),
        in_specs=[
            pl.BlockSpec((BT, D), lambda b, f, bex, nact: (b, 0)),
            pl.BlockSpec((1, FFB, D),
                         lambda b, f, bex, nact: (expert_of(b, bex, nact), f, 0)),
            pl.BlockSpec((1, FFB, D),
                         lambda b, f, bex, nact: (expert_of(b, bex, nact), f, 0)),
            pl.BlockSpec((1, D, FFB),
                         lambda b, f, bex, nact: (expert_of(b, bex, nact), 0, f)),
        ],
        out_specs=pl.BlockSpec((BT, D), lambda b, f, bex, nact: (b, 0)),
    )
    return pl.pallas_call(
        _gmm_body,
        grid_spec=grid_spec,
        out_shape=jax.ShapeDtypeStruct((P, D), jnp.float32),
        compiler_params=pltpu.CompilerParams(
            dimension_semantics=("arbitrary", "arbitrary")),
    )(bex, nact, xs, w1.astype(jnp.bfloat16), w3.astype(jnp.bfloat16),
      w2.astype(jnp.bfloat16))


def _combine_body(ys_hbm, p0_hbm, p1_hbm, w0_hbm, w1_hbm, out_hbm,
                  idx0_v, idx1_v, w0_v, w1_v, g0_v, g1_v, sem0, sem1):
    c = lax.axis_index("c")
    s = lax.axis_index("s")
    wid = s * 2 + c
    pltpu.sync_copy(p0_hbm.at[wid], idx0_v)
    pltpu.sync_copy(p1_hbm.at[wid], idx1_v)
    pltpu.sync_copy(w0_hbm.at[wid], w0_v)
    pltpu.sync_copy(w1_hbm.at[wid], w1_v)
    half_n = TPT // 2
    for half in range(2):
        cp0 = pltpu.async_copy(
            ys_hbm.at[idx0_v.at[pl.ds(half * half_n, half_n)]], g0_v, sem0)
        cp1 = pltpu.async_copy(
            ys_hbm.at[idx1_v.at[pl.ds(half * half_n, half_n)]], g1_v, sem1)
        cp0.wait()
        cp1.wait()

        def row_body(r, _, half=half):
            a = w0_v[half * half_n + r, :]
            bw = w1_v[half * half_n + r, :]
            for cc in range(D // 16):
                sl = pl.ds(cc * 16, 16)
                g0_v[r, sl] = a * g0_v[r, sl] + bw * g1_v[r, sl]
            return 0

        lax.fori_loop(0, half_n, row_body, 0)
        pltpu.sync_copy(g0_v, out_hbm.at[pl.ds(wid * TPT + half * half_n, half_n)])


def _combine(ys, p0, p1, w0m, w1m):
    mesh = plsc.VectorSubcoreMesh(core_axis_name="c", subcore_axis_name="s")
    half_n = TPT // 2
    fn = functools.partial(
        pl.kernel,
        out_type=jax.ShapeDtypeStruct((T, D), jnp.float32),
        mesh=mesh,
        scratch_types=[
            pltpu.VMEM((TPT,), jnp.int32),
            pltpu.VMEM((TPT,), jnp.int32),
            pltpu.VMEM((TPT, 16), jnp.float32),
            pltpu.VMEM((TPT, 16), jnp.float32),
            pltpu.VMEM((half_n, D), jnp.float32),
            pltpu.VMEM((half_n, D), jnp.float32),
            pltpu.SemaphoreType.DMA,
            pltpu.SemaphoreType.DMA,
        ],
    )(_combine_body)
    return fn(ys, p0, p1, w0m, w1m)


def kernel(hidden_states, gate_w, w1, w2, w3):
    b, s, d = hidden_states.shape
    x = hidden_states.reshape(-1, d)
    gw_pad = jnp.pad(gate_w, ((0, LANES - E), (0, 0)))
    pos0, pos1, w0c, w1c, meta = _routing(x, gw_pad)
    p0 = pos0.reshape(NTILES, TPT)
    p1 = pos1.reshape(NTILES, TPT)
    w0m = w0c.reshape(NTILES, TPT, 16)
    w1m = w1c.reshape(NTILES, TPT, 16)
    bex = meta[0, :NBMAX]
    nact = meta[0, NBMAX:NBMAX + 1]
    xs = _dispatch(x, p0, p1)
    ys = _gmm(bex, nact, xs, w1, w3, w2)
    out = _combine(ys, p0, p1, w0m, w1m)
    return out.reshape(b, s, d)


# speedup vs baseline: 1.2493x; 1.2493x over previous
import functools

import jax
import jax.numpy as jnp
from jax import lax
from jax.experimental import pallas as pl
from jax.experimental.pallas import tpu as pltpu
from jax.experimental.pallas import tpu_sc as plsc

E = 8
TOPK = 2
T = 2048
D = 1024
FF = 3584

BT = 256
NBMAX = 16 + E - 1
P = NBMAX * BT
FFB = 512
NF = FF // FFB

NTILES = 32
TPT = T // NTILES
LANES = 128


def _routing_body(x_ref, gw_ref, pos0_ref, pos1_ref, w0_ref, w1_ref, meta_ref):
    x = x_ref[...]
    gw = gw_ref[...]
    logits = lax.dot_general(x, gw, (((1,), (1,)), ((), ())),
                             preferred_element_type=jnp.float32)
    lane = lax.broadcasted_iota(jnp.int32, (T, LANES), 1)
    neg = jnp.float32(-1e30)
    logits = jnp.where(lane < E, logits, neg)

    m0 = jnp.max(logits, axis=1, keepdims=True)
    i0 = jnp.min(jnp.where(logits == m0, lane, LANES), axis=1, keepdims=True)
    l2 = jnp.where(lane == i0, neg, logits)
    m1 = jnp.max(l2, axis=1, keepdims=True)
    i1 = jnp.min(jnp.where(l2 == m1, lane, LANES), axis=1, keepdims=True)

    ex = jnp.exp(m1 - m0)
    w0 = 1.0 / (1.0 + ex)
    w1 = ex / (1.0 + ex)

    oh0 = (lane == i0).astype(jnp.float32)
    oh1 = (lane == i1).astype(jnp.float32)
    cnt = oh0 + oh1

    csum = cnt
    s = 1
    while s < T:
        csum = csum + jnp.concatenate(
            [jnp.zeros((s, LANES), jnp.float32), csum[:-s, :]], axis=0)
        s *= 2
    excl = csum - cnt
    counts = csum[T - 1:T, :]

    nb = jnp.floor((counts + (BT - 1)) / BT)
    nb = jnp.where(lane[:1, :] < E, nb, 0.0)
    pnb = nb
    s = 1
    while s < LANES:
        pnb = pnb + jnp.concatenate(
            [jnp.zeros((1, s), jnp.float32), pnb[:, :-s]], axis=1)
        s *= 2
    pext = pnb - nb
    padded_off = BT * pext

    slot = excl + padded_off
    pos0 = jnp.sum(oh0 * slot, axis=1, keepdims=True)
    pos1 = jnp.sum(oh1 * slot, axis=1, keepdims=True)

    nact = jnp.sum(jnp.where(lane[:1, :] == E - 1, pnb, 0.0),
                   axis=1, keepdims=True)
    bf = lane[:1, :].astype(jnp.float32)
    be = jnp.zeros((1, LANES), jnp.float32)
    for e in range(E):
        pnb_e = jnp.sum(jnp.where(lane[:1, :] == e, pnb, 0.0),
                        axis=1, keepdims=True)
        be = be + (bf >= pnb_e).astype(jnp.float32)
    be = jnp.minimum(be, float(E - 1))

    meta = jnp.where(lane[:1, :] < NBMAX, be,
                     jnp.where(lane[:1, :] == NBMAX, nact, 0.0))

    pos0_ref[...] = pos0.astype(jnp.int32)
    pos1_ref[...] = pos1.astype(jnp.int32)
    w0_ref[...] = jnp.broadcast_to(w0, (T, 16))
    w1_ref[...] = jnp.broadcast_to(w1, (T, 16))
    meta_ref[...] = meta.astype(jnp.int32)


def _routing(x, gw_pad):
    return pl.pallas_call(
        _routing_body,
        out_shape=[
            jax.ShapeDtypeStruct((T, 1), jnp.int32),
            jax.ShapeDtypeStruct((T, 1), jnp.int32),
            jax.ShapeDtypeStruct((T, 16), jnp.float32),
            jax.ShapeDtypeStruct((T, 16), jnp.float32),
            jax.ShapeDtypeStruct((1, LANES), jnp.int32),
        ],
    )(x, gw_pad)


def _dispatch_body(x_hbm, p0_hbm, p1_hbm, out_hbm, idx0_v, idx1_v, rows_v, sem):
    c = lax.axis_index("c")
    s = lax.axis_index("s")
    wid = s * 2 + c
    pltpu.sync_copy(p0_hbm.at[wid], idx0_v)
    pltpu.sync_copy(p1_hbm.at[wid], idx1_v)
    pltpu.sync_copy(x_hbm.at[pl.ds(wid * TPT, TPT)], rows_v)
    pltpu.async_copy(rows_v, out_hbm.at[idx0_v], sem).wait()
    pltpu.async_copy(rows_v, out_hbm.at[idx1_v], sem).wait()


def _dispatch(x, p0, p1):
    mesh = plsc.VectorSubcoreMesh(core_axis_name="c", subcore_axis_name="s")
    fn = functools.partial(
        pl.kernel,
        out_type=jax.ShapeDtypeStruct((P, D), jnp.float32),
        mesh=mesh,
        scratch_types=[
            pltpu.VMEM((TPT,), jnp.int32),
            pltpu.VMEM((TPT,), jnp.int32),
            pltpu.VMEM((TPT, D), jnp.float32),
            pltpu.SemaphoreType.DMA,
        ],
    )(_dispatch_body)
    return fn(x, p0, p1)


def _gmm_body(bex_ref, nact_ref, xs_ref, w1_ref, w3_ref, w2_ref, out_ref,
              acc_ref, sem):
    f = pl.program_id(0)
    j = pl.program_id(1)

    @pl.when(j < nact_ref[0])
    def _():
        x = xs_ref[...]
        h1 = lax.dot_general(x, w1_ref[0], (((1,), (1,)), ((), ())),
                             preferred_element_type=jnp.float32)
        h3 = lax.dot_general(x, w3_ref[0], (((1,), (1,)), ((), ())),
                             preferred_element_type=jnp.float32)
        h = h1 * lax.logistic(h1) * h3
        y = lax.dot_general(h, w2_ref[0], (((1,), (1,)), ((), ())),
                            preferred_element_type=jnp.float32)

        @pl.when(f == 0)
        def _():
            acc_ref[pl.ds(j, 1)] = y[None]

        @pl.when(f > 0)
        def _():
            acc_ref[pl.ds(j, 1)] += y[None]

        @pl.when(f == NF - 1)
        def _():
            cp = pltpu.make_async_copy(
                acc_ref.at[j], out_ref.at[pl.ds(j * BT, BT)], sem)
            cp.start()
            cp.wait()


def _gmm(bex, nact, xs, w1, w3, w2):
    def expert_of(j, bex_ref, nact_ref):
        return bex_ref[jnp.minimum(j, nact_ref[0] - 1)]

    grid_spec = pltpu.PrefetchScalarGridSpec(
        num_scalar_prefetch=2,
        grid=(NF, NBMAX),
        in_specs=[
            pl.BlockSpec((BT, D), lambda f, j, bex, nact: (j, 0)),
            pl.BlockSpec((1, FFB, D),
                         lambda f, j, bex, nact: (expert_of(j, bex, nact), f, 0)),
            pl.BlockSpec((1, FFB, D),
                         lambda f, j, bex, nact: (expert_of(j, bex, nact), f, 0)),
            pl.BlockSpec((1, D, FFB),
                         lambda f, j, bex, nact: (expert_of(j, bex, nact), 0, f)),
        ],
        out_specs=pl.BlockSpec(memory_space=pl.ANY),
        scratch_shapes=[
            pltpu.VMEM((NBMAX, BT, D), jnp.float32),
            pltpu.SemaphoreType.DMA,
        ],
    )
    return pl.pallas_call(
        _gmm_body,
        grid_spec=grid_spec,
        out_shape=jax.ShapeDtypeStruct((P, D), jnp.float32),
        compiler_params=pltpu.CompilerParams(
            dimension_semantics=("arbitrary", "arbitrary")),
    )(bex, nact, xs, w1, w3, w2)


def _combine_body(ys_hbm, p0_hbm, p1_hbm, w0_hbm, w1_hbm, out_hbm,
                  idx0_v, idx1_v, w0_v, w1_v, g0_v, g1_v, sem0, sem1):
    c = lax.axis_index("c")
    s = lax.axis_index("s")
    wid = s * 2 + c
    pltpu.sync_copy(p0_hbm.at[wid], idx0_v)
    pltpu.sync_copy(p1_hbm.at[wid], idx1_v)
    pltpu.sync_copy(w0_hbm.at[wid], w0_v)
    pltpu.sync_copy(w1_hbm.at[wid], w1_v)
    half_n = TPT // 2
    for half in range(2):
        cp0 = pltpu.async_copy(
            ys_hbm.at[idx0_v.at[pl.ds(half * half_n, half_n)]], g0_v, sem0)
        cp1 = pltpu.async_copy(
            ys_hbm.at[idx1_v.at[pl.ds(half * half_n, half_n)]], g1_v, sem1)
        cp0.wait()
        cp1.wait()

        def row_body(r, _, half=half):
            a = w0_v[half * half_n + r, :]
            bw = w1_v[half * half_n + r, :]
            for cc in range(D // 16):
                sl = pl.ds(cc * 16, 16)
                g0_v[r, sl] = a * g0_v[r, sl] + bw * g1_v[r, sl]
            return 0

        lax.fori_loop(0, half_n, row_body, 0)
        pltpu.sync_copy(g0_v, out_hbm.at[pl.ds(wid * TPT + half * half_n, half_n)])


def _combine(ys, p0, p1, w0m, w1m):
    mesh = plsc.VectorSubcoreMesh(core_axis_name="c", subcore_axis_name="s")
    half_n = TPT // 2
    fn = functools.partial(
        pl.kernel,
        out_type=jax.ShapeDtypeStruct((T, D), jnp.float32),
        mesh=mesh,
        scratch_types=[
            pltpu.VMEM((TPT,), jnp.int32),
            pltpu.VMEM((TPT,), jnp.int32),
            pltpu.VMEM((TPT, 16), jnp.float32),
            pltpu.VMEM((TPT, 16), jnp.float32),
            pltpu.VMEM((half_n, D), jnp.float32),
            pltpu.VMEM((half_n, D), jnp.float32),
            pltpu.SemaphoreType.DMA,
            pltpu.SemaphoreType.DMA,
        ],
    )(_combine_body)
    return fn(ys, p0, p1, w0m, w1m)


def kernel(hidden_states, gate_w, w1, w2, w3):
    b, s, d = hidden_states.shape
    x = hidden_states.reshape(-1, d)
    gw_pad = jnp.pad(gate_w, ((0, LANES - E), (0, 0)))
    pos0, pos1, w0c, w1c, meta = _routing(x, gw_pad)
    p0 = pos0.reshape(NTILES, TPT)
    p1 = pos1.reshape(NTILES, TPT)
    w0m = w0c.reshape(NTILES, TPT, 16)
    w1m = w1c.reshape(NTILES, TPT, 16)
    bex = meta[0, :NBMAX]
    nact = meta[0, NBMAX:NBMAX + 1]
    xs = _dispatch(x, p0, p1)
    ys = _gmm(bex, nact, xs, w1, w3, w2)
    out = _combine(ys, p0, p1, w0m, w1m)
    return out.reshape(b, s, d)


# speedup vs baseline: 1.5484x; 1.2395x over previous
import functools

import jax
import jax.numpy as jnp
from jax import lax
from jax.experimental import pallas as pl
from jax.experimental.pallas import tpu as pltpu
from jax.experimental.pallas import tpu_sc as plsc

E = 8
TOPK = 2
T = 2048
D = 1024
FF = 3584

BT = 256
NBMAX = 16 + E - 1
P = NBMAX * BT
FFB = 512
NF = FF // FFB

NTILES = 32
TPT = T // NTILES
LANES = 128


def _routing_body(x_ref, gw_ref, pos0_ref, pos1_ref, w0_ref, w1_ref, meta_ref):
    x = x_ref[...]
    gw = gw_ref[...]
    logits = lax.dot_general(x, gw, (((1,), (1,)), ((), ())),
                             preferred_element_type=jnp.float32)
    lane = lax.broadcasted_iota(jnp.int32, (T, LANES), 1)
    neg = jnp.float32(-1e30)
    logits = jnp.where(lane < E, logits, neg)

    m0 = jnp.max(logits, axis=1, keepdims=True)
    i0 = jnp.min(jnp.where(logits == m0, lane, LANES), axis=1, keepdims=True)
    l2 = jnp.where(lane == i0, neg, logits)
    m1 = jnp.max(l2, axis=1, keepdims=True)
    i1 = jnp.min(jnp.where(l2 == m1, lane, LANES), axis=1, keepdims=True)

    ex = jnp.exp(m1 - m0)
    w0 = 1.0 / (1.0 + ex)
    w1 = ex / (1.0 + ex)

    oh0 = (lane == i0).astype(jnp.float32)
    oh1 = (lane == i1).astype(jnp.float32)
    cnt = oh0 + oh1

    csum = cnt
    s = 1
    while s < T:
        csum = csum + jnp.concatenate(
            [jnp.zeros((s, LANES), jnp.float32), csum[:-s, :]], axis=0)
        s *= 2
    excl = csum - cnt
    counts = csum[T - 1:T, :]

    nb = jnp.floor((counts + (BT - 1)) / BT)
    nb = jnp.where(lane[:1, :] < E, nb, 0.0)
    pnb = nb
    s = 1
    while s < LANES:
        pnb = pnb + jnp.concatenate(
            [jnp.zeros((1, s), jnp.float32), pnb[:, :-s]], axis=1)
        s *= 2
    pext = pnb - nb
    padded_off = BT * pext

    slot = excl + padded_off
    pos0 = jnp.sum(oh0 * slot, axis=1, keepdims=True)
    pos1 = jnp.sum(oh1 * slot, axis=1, keepdims=True)

    nact = jnp.sum(jnp.where(lane[:1, :] == E - 1, pnb, 0.0),
                   axis=1, keepdims=True)
    bf = lane[:1, :].astype(jnp.float32)
    be = jnp.zeros((1, LANES), jnp.float32)
    for e in range(E):
        pnb_e = jnp.sum(jnp.where(lane[:1, :] == e, pnb, 0.0),
                        axis=1, keepdims=True)
        be = be + (bf >= pnb_e).astype(jnp.float32)
    be = jnp.minimum(be, float(E - 1))

    be_prev = jnp.concatenate([be[:, :1] - 1.0, be[:, :NBMAX - 1],
                               jnp.zeros((1, LANES - NBMAX), jnp.float32)],
                              axis=1)
    sta = jnp.where(lane[:1, :] < NBMAX,
                    (be != be_prev).astype(jnp.float32), 0.0)
    rid = sta
    s = 1
    while s < LANES:
        rid = rid + jnp.concatenate(
            [jnp.zeros((1, s), jnp.float32), rid[:, :-s]], axis=1)
        s *= 2
    rid = rid - 1.0
    rpf = jnp.sum(jnp.where(lane[:1, :] == NBMAX - 1, rid + 1.0, 0.0),
                  axis=1, keepdims=True)
    present = (nb > 0.0).astype(jnp.float32)
    lanef = lane[:1, :].astype(jnp.float32)
    nxe = jnp.zeros((1, LANES), jnp.float32)
    for v in range(E):
        m_v = jnp.min(jnp.where((lane[:1, :] < E) & (lane[:1, :] > v)
                                & (present > 0.0), lanef, float(E - 1)),
                      axis=1, keepdims=True)
        nxe = nxe + (be == v).astype(jnp.float32) * m_v

    meta = jnp.concatenate([be, nxe, rid, sta,
                            jnp.broadcast_to(nact, (1, LANES)),
                            jnp.broadcast_to(rpf, (1, LANES))], axis=0)

    pos0_ref[...] = pos0.astype(jnp.int32)
    pos1_ref[...] = pos1.astype(jnp.int32)
    w0_ref[...] = jnp.broadcast_to(w0, (T, 16))
    w1_ref[...] = jnp.broadcast_to(w1, (T, 16))
    meta_ref[...] = meta.astype(jnp.int32)


def _routing(x, gw_pad):
    return pl.pallas_call(
        _routing_body,
        out_shape=[
            jax.ShapeDtypeStruct((T, 1), jnp.int32),
            jax.ShapeDtypeStruct((T, 1), jnp.int32),
            jax.ShapeDtypeStruct((T, 16), jnp.float32),
            jax.ShapeDtypeStruct((T, 16), jnp.float32),
            jax.ShapeDtypeStruct((6, LANES), jnp.int32),
        ],
    )(x, gw_pad)


def _dispatch_body(x_hbm, p0_hbm, p1_hbm, out_hbm, idx0_v, idx1_v, rows_v, sem):
    c = lax.axis_index("c")
    s = lax.axis_index("s")
    wid = s * 2 + c
    pltpu.sync_copy(p0_hbm.at[wid], idx0_v)
    pltpu.sync_copy(p1_hbm.at[wid], idx1_v)
    pltpu.sync_copy(x_hbm.at[pl.ds(wid * TPT, TPT)], rows_v)
    pltpu.async_copy(rows_v, out_hbm.at[idx0_v], sem).wait()
    pltpu.async_copy(rows_v, out_hbm.at[idx1_v], sem).wait()


def _dispatch(x, p0, p1):
    mesh = plsc.VectorSubcoreMesh(core_axis_name="c", subcore_axis_name="s")
    fn = functools.partial(
        pl.kernel,
        out_type=jax.ShapeDtypeStruct((P, D), jnp.float32),
        mesh=mesh,
        scratch_types=[
            pltpu.VMEM((TPT,), jnp.int32),
            pltpu.VMEM((TPT,), jnp.int32),
            pltpu.VMEM((TPT, D), jnp.float32),
            pltpu.SemaphoreType.DMA,
        ],
    )(_dispatch_body)
    return fn(x, p0, p1)


def _gmm_body(bex_ref, nxe_ref, rid_ref, sta_ref, nact_ref, rpf_ref,
              xs_ref, w1_hbm, w3_hbm, w2_hbm, out_ref,
              acc_ref, wb1, wb3, wb2, s1, s3, s2, osem):
    f = pl.program_id(0)
    j = pl.program_id(1)
    ecur = bex_ref[j]
    r = rid_ref[j]
    parity = lax.rem(f * rpf_ref[0] + r, 2)

    def copies(e, fidx, slot):
        return (
            pltpu.make_async_copy(
                w1_hbm.at[e, pl.ds(fidx * FFB, FFB)], wb1.at[slot], s1.at[slot]),
            pltpu.make_async_copy(
                w3_hbm.at[e, pl.ds(fidx * FFB, FFB)], wb3.at[slot], s3.at[slot]),
            pltpu.make_async_copy(
                w2_hbm.at[e, :, pl.ds(fidx * FFB, FFB)], wb2.at[slot], s2.at[slot]),
        )

    @pl.when(sta_ref[j] == 1)
    def _():
        @pl.when((f == 0) & (j == 0))
        def _():
            for cp in copies(ecur, f, parity):
                cp.start()
        for cp in copies(ecur, f, parity):
            cp.wait()
        islast = r == rpf_ref[0] - 1
        e_n = jnp.where(islast, bex_ref[0], nxe_ref[j])
        f_n = jnp.where(islast, f + 1, f)

        @pl.when(jnp.logical_not((f == NF - 1) & islast))
        def _():
            for cp in copies(e_n, f_n, 1 - parity):
                cp.start()

    @pl.when(j < nact_ref[0])
    def _():
        x = xs_ref[...]
        w1c = wb1[parity]
        w3c = wb3[parity]
        w2c = wb2[parity]
        h1 = lax.dot_general(x, w1c, (((1,), (1,)), ((), ())),
                             preferred_element_type=jnp.float32)
        h3 = lax.dot_general(x, w3c, (((1,), (1,)), ((), ())),
                             preferred_element_type=jnp.float32)
        h = h1 * lax.logistic(h1) * h3
        y = lax.dot_general(h, w2c, (((1,), (1,)), ((), ())),
                            preferred_element_type=jnp.float32)

        @pl.when(f == 0)
        def _():
            acc_ref[pl.ds(j, 1)] = y[None]

        @pl.when(f > 0)
        def _():
            acc_ref[pl.ds(j, 1)] += y[None]

        @pl.when(f == NF - 1)
        def _():
            cp = pltpu.make_async_copy(
                acc_ref.at[j], out_ref.at[pl.ds(j * BT, BT)], osem)
            cp.start()
            cp.wait()


def _gmm(bex, nxe, rid, sta, nact, rpf, xs, w1, w3, w2):
    grid_spec = pltpu.PrefetchScalarGridSpec(
        num_scalar_prefetch=6,
        grid=(NF, NBMAX),
        in_specs=[
            pl.BlockSpec((BT, D), lambda f, j, *_: (j, 0)),
            pl.BlockSpec(memory_space=pl.ANY),
            pl.BlockSpec(memory_space=pl.ANY),
            pl.BlockSpec(memory_space=pl.ANY),
        ],
        out_specs=pl.BlockSpec(memory_space=pl.ANY),
        scratch_shapes=[
            pltpu.VMEM((NBMAX, BT, D), jnp.float32),
            pltpu.VMEM((2, FFB, D), jnp.float32),
            pltpu.VMEM((2, FFB, D), jnp.float32),
            pltpu.VMEM((2, D, FFB), jnp.float32),
            pltpu.SemaphoreType.DMA((2,)),
            pltpu.SemaphoreType.DMA((2,)),
            pltpu.SemaphoreType.DMA((2,)),
            pltpu.SemaphoreType.DMA,
        ],
    )
    return pl.pallas_call(
        _gmm_body,
        grid_spec=grid_spec,
        out_shape=jax.ShapeDtypeStruct((P, D), jnp.float32),
        compiler_params=pltpu.CompilerParams(
            dimension_semantics=("arbitrary", "arbitrary")),
    )(bex, nxe, rid, sta, nact, rpf, xs, w1, w3, w2)


def _combine_body(ys_hbm, p0_hbm, p1_hbm, w0_hbm, w1_hbm, out_hbm,
                  idx0_v, idx1_v, w0_v, w1_v, g0_v, g1_v, sem0, sem1):
    c = lax.axis_index("c")
    s = lax.axis_index("s")
    wid = s * 2 + c
    pltpu.sync_copy(p0_hbm.at[wid], idx0_v)
    pltpu.sync_copy(p1_hbm.at[wid], idx1_v)
    pltpu.sync_copy(w0_hbm.at[wid], w0_v)
    pltpu.sync_copy(w1_hbm.at[wid], w1_v)
    half_n = TPT // 2
    for half in range(2):
        cp0 = pltpu.async_copy(
            ys_hbm.at[idx0_v.at[pl.ds(half * half_n, half_n)]], g0_v, sem0)
        cp1 = pltpu.async_copy(
            ys_hbm.at[idx1_v.at[pl.ds(half * half_n, half_n)]], g1_v, sem1)
        cp0.wait()
        cp1.wait()

        def row_body(r, _, half=half):
            a = w0_v[half * half_n + r, :]
            bw = w1_v[half * half_n + r, :]
            for cc in range(D // 16):
                sl = pl.ds(cc * 16, 16)
                g0_v[r, sl] = a * g0_v[r, sl] + bw * g1_v[r, sl]
            return 0

        lax.fori_loop(0, half_n, row_body, 0)
        pltpu.sync_copy(g0_v, out_hbm.at[pl.ds(wid * TPT + half * half_n, half_n)])


def _combine(ys, p0, p1, w0m, w1m):
    mesh = plsc.VectorSubcoreMesh(core_axis_name="c", subcore_axis_name="s")
    half_n = TPT // 2
    fn = functools.partial(
        pl.kernel,
        out_type=jax.ShapeDtypeStruct((T, D), jnp.float32),
        mesh=mesh,
        scratch_types=[
            pltpu.VMEM((TPT,), jnp.int32),
            pltpu.VMEM((TPT,), jnp.int32),
            pltpu.VMEM((TPT, 16), jnp.float32),
            pltpu.VMEM((TPT, 16), jnp.float32),
            pltpu.VMEM((half_n, D), jnp.float32),
            pltpu.VMEM((half_n, D), jnp.float32),
            pltpu.SemaphoreType.DMA,
            pltpu.SemaphoreType.DMA,
        ],
    )(_combine_body)
    return fn(ys, p0, p1, w0m, w1m)


def kernel(hidden_states, gate_w, w1, w2, w3):
    b, s, d = hidden_states.shape
    x = hidden_states.reshape(-1, d)
    gw_pad = jnp.pad(gate_w, ((0, LANES - E), (0, 0)))
    pos0, pos1, w0c, w1c, meta = _routing(x, gw_pad)
    p0 = pos0.reshape(NTILES, TPT)
    p1 = pos1.reshape(NTILES, TPT)
    w0m = w0c.reshape(NTILES, TPT, 16)
    w1m = w1c.reshape(NTILES, TPT, 16)
    bex = meta[0, :NBMAX]
    nxe = meta[1, :NBMAX]
    rid = meta[2, :NBMAX]
    sta = meta[3, :NBMAX]
    nact = meta[4, :1]
    rpf = meta[5, :1]
    xs = _dispatch(x, p0, p1)
    ys = _gmm(bex, nxe, rid, sta, nact, rpf, xs, w1, w3, w2)
    out = _combine(ys, p0, p1, w0m, w1m)
    return out.reshape(b, s, d)


# speedup vs baseline: 1.5509x; 1.0016x over previous
import functools

import jax
import jax.numpy as jnp
from jax import lax
from jax.experimental import pallas as pl
from jax.experimental.pallas import tpu as pltpu
from jax.experimental.pallas import tpu_sc as plsc

E = 8
TOPK = 2
T = 2048
D = 1024
FF = 3584

BT = 256
NBMAX = (2 * T) // BT + E - 1
P = NBMAX * BT
FFB = 512
NF = FF // FFB

NTILES = 32
TPT = T // NTILES
LANES = 128


def _routing_body(x_ref, gw_ref, pos0_ref, pos1_ref, w0_ref, w1_ref, meta_ref):
    x = x_ref[...]
    gw = gw_ref[...]
    logits = lax.dot_general(x, gw, (((1,), (1,)), ((), ())),
                             preferred_element_type=jnp.float32)
    lane = lax.broadcasted_iota(jnp.int32, (T, LANES), 1)
    neg = jnp.float32(-1e30)
    logits = jnp.where(lane < E, logits, neg)

    m0 = jnp.max(logits, axis=1, keepdims=True)
    i0 = jnp.min(jnp.where(logits == m0, lane, LANES), axis=1, keepdims=True)
    l2 = jnp.where(lane == i0, neg, logits)
    m1 = jnp.max(l2, axis=1, keepdims=True)
    i1 = jnp.min(jnp.where(l2 == m1, lane, LANES), axis=1, keepdims=True)

    ex = jnp.exp(m1 - m0)
    w0 = 1.0 / (1.0 + ex)
    w1 = ex / (1.0 + ex)

    oh0 = (lane == i0).astype(jnp.float32)
    oh1 = (lane == i1).astype(jnp.float32)
    cnt = oh0 + oh1

    csum = cnt
    s = 1
    while s < T:
        csum = csum + jnp.concatenate(
            [jnp.zeros((s, LANES), jnp.float32), csum[:-s, :]], axis=0)
        s *= 2
    excl = csum - cnt
    counts = csum[T - 1:T, :]

    nb = jnp.floor((counts + (BT - 1)) / BT)
    nb = jnp.where(lane[:1, :] < E, nb, 0.0)
    pnb = nb
    s = 1
    while s < LANES:
        pnb = pnb + jnp.concatenate(
            [jnp.zeros((1, s), jnp.float32), pnb[:, :-s]], axis=1)
        s *= 2
    pext = pnb - nb
    padded_off = BT * pext

    slot = excl + padded_off
    pos0 = jnp.sum(oh0 * slot, axis=1, keepdims=True)
    pos1 = jnp.sum(oh1 * slot, axis=1, keepdims=True)

    nact = jnp.sum(jnp.where(lane[:1, :] == E - 1, pnb, 0.0),
                   axis=1, keepdims=True)
    bf = lane[:1, :].astype(jnp.float32)
    be = jnp.zeros((1, LANES), jnp.float32)
    for e in range(E):
        pnb_e = jnp.sum(jnp.where(lane[:1, :] == e, pnb, 0.0),
                        axis=1, keepdims=True)
        be = be + (bf >= pnb_e).astype(jnp.float32)
    be = jnp.minimum(be, float(E - 1))

    be_prev = jnp.concatenate([be[:, :1] - 1.0, be[:, :NBMAX - 1],
                               jnp.zeros((1, LANES - NBMAX), jnp.float32)],
                              axis=1)
    sta = jnp.where(lane[:1, :] < NBMAX,
                    (be != be_prev).astype(jnp.float32), 0.0)
    rid = sta
    s = 1
    while s < LANES:
        rid = rid + jnp.concatenate(
            [jnp.zeros((1, s), jnp.float32), rid[:, :-s]], axis=1)
        s *= 2
    rid = rid - 1.0
    rpf = jnp.sum(jnp.where(lane[:1, :] == NBMAX - 1, rid + 1.0, 0.0),
                  axis=1, keepdims=True)
    rex = jnp.zeros((1, LANES), jnp.float32)
    for r in range(E):
        rex_r = jnp.min(jnp.where((rid == r) & (lane[:1, :] < NBMAX),
                                  be, float(E)), axis=1, keepdims=True)
        rex = rex + (lane[:1, :] == r).astype(jnp.float32) * rex_r
    rex = jnp.minimum(rex, float(E - 1))

    meta = jnp.concatenate([be, rex, rid, sta,
                            jnp.broadcast_to(nact, (1, LANES)),
                            jnp.broadcast_to(rpf, (1, LANES))], axis=0)

    pos0_ref[...] = pos0.astype(jnp.int32)
    pos1_ref[...] = pos1.astype(jnp.int32)
    w0_ref[...] = jnp.broadcast_to(w0, (T, 16))
    w1_ref[...] = jnp.broadcast_to(w1, (T, 16))
    meta_ref[...] = meta.astype(jnp.int32)


def _routing(x, gw_pad):
    return pl.pallas_call(
        _routing_body,
        out_shape=[
            jax.ShapeDtypeStruct((T, 1), jnp.int32),
            jax.ShapeDtypeStruct((T, 1), jnp.int32),
            jax.ShapeDtypeStruct((T, 16), jnp.float32),
            jax.ShapeDtypeStruct((T, 16), jnp.float32),
            jax.ShapeDtypeStruct((6, LANES), jnp.int32),
        ],
    )(x, gw_pad)


def _dispatch_body(x_hbm, p0_hbm, p1_hbm, out_hbm, idx0_v, idx1_v, rows_v, sem):
    c = lax.axis_index("c")
    s = lax.axis_index("s")
    wid = s * 2 + c
    pltpu.sync_copy(p0_hbm.at[wid], idx0_v)
    pltpu.sync_copy(p1_hbm.at[wid], idx1_v)
    pltpu.sync_copy(x_hbm.at[pl.ds(wid * TPT, TPT)], rows_v)
    pltpu.async_copy(rows_v, out_hbm.at[idx0_v], sem).wait()
    pltpu.async_copy(rows_v, out_hbm.at[idx1_v], sem).wait()


def _dispatch(x, p0, p1):
    mesh = plsc.VectorSubcoreMesh(core_axis_name="c", subcore_axis_name="s")
    fn = functools.partial(
        pl.kernel,
        out_type=jax.ShapeDtypeStruct((P, D), jnp.float32),
        mesh=mesh,
        scratch_types=[
            pltpu.VMEM((TPT,), jnp.int32),
            pltpu.VMEM((TPT,), jnp.int32),
            pltpu.VMEM((TPT, D), jnp.float32),
            pltpu.SemaphoreType.DMA,
        ],
    )(_dispatch_body)
    return fn(x, p0, p1)


NSLOT = 3


def _gmm_body(bex_ref, rex_ref, rid_ref, sta_ref, nact_ref, rpf_ref,
              xs_ref, w1_hbm, w3_hbm, w2_hbm, out_ref,
              acc_ref, wb1, wb3, wb2, s1, s3, s2, osem):
    f = pl.program_id(0)
    j = pl.program_id(1)
    ecur = bex_ref[j]
    r = rid_ref[j]
    rpf = rpf_ref[0]
    g = f * rpf + r
    slot = lax.rem(g, NSLOT)

    def copies(e, fidx, sl):
        return (
            pltpu.make_async_copy(
                w1_hbm.at[e, pl.ds(fidx * FFB, FFB)], wb1.at[sl], s1.at[sl]),
            pltpu.make_async_copy(
                w3_hbm.at[e, pl.ds(fidx * FFB, FFB)], wb3.at[sl], s3.at[sl]),
            pltpu.make_async_copy(
                w2_hbm.at[e, :, pl.ds(fidx * FFB, FFB)], wb2.at[sl], s2.at[sl]),
        )

    def issue_run(gt):
        f_t = lax.div(gt, rpf)
        r_t = gt - f_t * rpf

        @pl.when(f_t < NF)
        def _():
            for cp in copies(rex_ref[r_t], f_t, lax.rem(gt, NSLOT)):
                cp.start()

    @pl.when(sta_ref[j] == 1)
    def _():
        @pl.when((f == 0) & (j == 0))
        def _():
            issue_run(0)
            issue_run(1)
        for cp in copies(ecur, f, slot):
            cp.wait()
        issue_run(g + 2)

    @pl.when(j < nact_ref[0])
    def _():
        x = xs_ref[...]
        w1c = wb1[slot]
        w3c = wb3[slot]
        w2c = wb2[slot]
        h1 = lax.dot_general(x, w1c, (((1,), (1,)), ((), ())),
                             preferred_element_type=jnp.float32)
        h3 = lax.dot_general(x, w3c, (((1,), (1,)), ((), ())),
                             preferred_element_type=jnp.float32)
        h = h1 * lax.logistic(h1) * h3
        y = lax.dot_general(h, w2c, (((1,), (1,)), ((), ())),
                            preferred_element_type=jnp.float32)

        @pl.when(f == 0)
        def _():
            acc_ref[pl.ds(j, 1)] = y[None]

        @pl.when(f > 0)
        def _():
            acc_ref[pl.ds(j, 1)] += y[None]

        @pl.when(f == NF - 1)
        def _():
            cp = pltpu.make_async_copy(
                acc_ref.at[j], out_ref.at[pl.ds(j * BT, BT)], osem)
            cp.start()
            cp.wait()


def _gmm(bex, rex, rid, sta, nact, rpf, xs, w1, w3, w2):
    grid_spec = pltpu.PrefetchScalarGridSpec(
        num_scalar_prefetch=6,
        grid=(NF, NBMAX),
        in_specs=[
            pl.BlockSpec((BT, D), lambda f, j, *_: (j, 0)),
            pl.BlockSpec(memory_space=pl.ANY),
            pl.BlockSpec(memory_space=pl.ANY),
            pl.BlockSpec(memory_space=pl.ANY),
        ],
        out_specs=pl.BlockSpec(memory_space=pl.ANY),
        scratch_shapes=[
            pltpu.VMEM((NBMAX, BT, D), jnp.float32),
            pltpu.VMEM((NSLOT, FFB, D), jnp.float32),
            pltpu.VMEM((NSLOT, FFB, D), jnp.float32),
            pltpu.VMEM((NSLOT, D, FFB), jnp.float32),
            pltpu.SemaphoreType.DMA((NSLOT,)),
            pltpu.SemaphoreType.DMA((NSLOT,)),
            pltpu.SemaphoreType.DMA((NSLOT,)),
            pltpu.SemaphoreType.DMA,
        ],
    )
    return pl.pallas_call(
        _gmm_body,
        grid_spec=grid_spec,
        out_shape=jax.ShapeDtypeStruct((P, D), jnp.float32),
        compiler_params=pltpu.CompilerParams(
            dimension_semantics=("arbitrary", "arbitrary")),
    )(bex, rex, rid, sta, nact, rpf, xs, w1, w3, w2)


def _combine_body(ys_hbm, p0_hbm, p1_hbm, w0_hbm, w1_hbm, out_hbm,
                  idx0_v, idx1_v, w0_v, w1_v, g0_v, g1_v, sem0, sem1):
    c = lax.axis_index("c")
    s = lax.axis_index("s")
    wid = s * 2 + c
    pltpu.sync_copy(p0_hbm.at[wid], idx0_v)
    pltpu.sync_copy(p1_hbm.at[wid], idx1_v)
    pltpu.sync_copy(w0_hbm.at[wid], w0_v)
    pltpu.sync_copy(w1_hbm.at[wid], w1_v)
    half_n = TPT // 2
    for half in range(2):
        cp0 = pltpu.async_copy(
            ys_hbm.at[idx0_v.at[pl.ds(half * half_n, half_n)]], g0_v, sem0)
        cp1 = pltpu.async_copy(
            ys_hbm.at[idx1_v.at[pl.ds(half * half_n, half_n)]], g1_v, sem1)
        cp0.wait()
        cp1.wait()

        def row_body(r, _, half=half):
            a = w0_v[half * half_n + r, :]
            bw = w1_v[half * half_n + r, :]
            for cc in range(D // 16):
                sl = pl.ds(cc * 16, 16)
                g0_v[r, sl] = a * g0_v[r, sl] + bw * g1_v[r, sl]
            return 0

        lax.fori_loop(0, half_n, row_body, 0)
        pltpu.sync_copy(g0_v, out_hbm.at[pl.ds(wid * TPT + half * half_n, half_n)])


def _combine(ys, p0, p1, w0m, w1m):
    mesh = plsc.VectorSubcoreMesh(core_axis_name="c", subcore_axis_name="s")
    half_n = TPT // 2
    fn = functools.partial(
        pl.kernel,
        out_type=jax.ShapeDtypeStruct((T, D), jnp.float32),
        mesh=mesh,
        scratch_types=[
            pltpu.VMEM((TPT,), jnp.int32),
            pltpu.VMEM((TPT,), jnp.int32),
            pltpu.VMEM((TPT, 16), jnp.float32),
            pltpu.VMEM((TPT, 16), jnp.float32),
            pltpu.VMEM((half_n, D), jnp.float32),
            pltpu.VMEM((half_n, D), jnp.float32),
            pltpu.SemaphoreType.DMA,
            pltpu.SemaphoreType.DMA,
        ],
    )(_combine_body)
    return fn(ys, p0, p1, w0m, w1m)


def kernel(hidden_states, gate_w, w1, w2, w3):
    b, s, d = hidden_states.shape
    x = hidden_states.reshape(-1, d)
    gw_pad = jnp.pad(gate_w, ((0, LANES - E), (0, 0)))
    pos0, pos1, w0c, w1c, meta = _routing(x, gw_pad)
    p0 = pos0.reshape(NTILES, TPT)
    p1 = pos1.reshape(NTILES, TPT)
    w0m = w0c.reshape(NTILES, TPT, 16)
    w1m = w1c.reshape(NTILES, TPT, 16)
    bex = meta[0, :NBMAX]
    rex = meta[1, :E]
    rid = meta[2, :NBMAX]
    sta = meta[3, :NBMAX]
    nact = meta[4, :1]
    rpf = meta[5, :1]
    xs = _dispatch(x, p0, p1)
    ys = _gmm(bex, rex, rid, sta, nact, rpf, xs, w1, w3, w2)
    out = _combine(ys, p0, p1, w0m, w1m)
    return out.reshape(b, s, d)


# speedup vs baseline: 1.7289x; 1.1148x over previous
import functools

import jax
import jax.numpy as jnp
from jax import lax
from jax.experimental import pallas as pl
from jax.experimental.pallas import tpu as pltpu
from jax.experimental.pallas import tpu_sc as plsc

E = 8
TOPK = 2
T = 2048
D = 1024
FF = 3584

BT = 256
NBMAX = (2 * T) // BT + E - 1
P8 = 2 * T + 8 * (E - 1)
PX = P8 + BT
FFB = 512
NF = FF // FFB

NTILES = 32
TPT = T // NTILES
LANES = 128


def _routing_body(x_ref, gw_ref, pos0_ref, pos1_ref, w0_ref, w1_ref, meta_ref):
    x = x_ref[...]
    gw = gw_ref[...]
    logits = lax.dot_general(x, gw, (((1,), (1,)), ((), ())),
                             preferred_element_type=jnp.float32)
    lane = lax.broadcasted_iota(jnp.int32, (T, LANES), 1)
    neg = jnp.float32(-1e30)
    logits = jnp.where(lane < E, logits, neg)

    m0 = jnp.max(logits, axis=1, keepdims=True)
    i0 = jnp.min(jnp.where(logits == m0, lane, LANES), axis=1, keepdims=True)
    l2 = jnp.where(lane == i0, neg, logits)
    m1 = jnp.max(l2, axis=1, keepdims=True)
    i1 = jnp.min(jnp.where(l2 == m1, lane, LANES), axis=1, keepdims=True)

    ex = jnp.exp(m1 - m0)
    w0 = 1.0 / (1.0 + ex)
    w1 = ex / (1.0 + ex)

    oh0 = (lane == i0).astype(jnp.float32)
    oh1 = (lane == i1).astype(jnp.float32)
    cnt = oh0 + oh1

    csum = cnt
    s = 1
    while s < T:
        csum = csum + jnp.concatenate(
            [jnp.zeros((s, LANES), jnp.float32), csum[:-s, :]], axis=0)
        s *= 2
    excl = csum - cnt
    counts = csum[T - 1:T, :]

    nb = jnp.floor((counts + (BT - 1)) / BT)
    nb = jnp.where(lane[:1, :] < E, nb, 0.0)
    pnb = nb
    s = 1
    while s < LANES:
        pnb = pnb + jnp.concatenate(
            [jnp.zeros((1, s), jnp.float32), pnb[:, :-s]], axis=1)
        s *= 2
    pext = pnb - nb

    r8 = 8.0 * jnp.floor((counts + 7.0) / 8.0)
    r8 = jnp.where(lane[:1, :] < E, r8, 0.0)
    po8 = r8
    s = 1
    while s < LANES:
        po8 = po8 + jnp.concatenate(
            [jnp.zeros((1, s), jnp.float32), po8[:, :-s]], axis=1)
        s *= 2
    off8 = po8 - r8

    slot = excl + off8
    pos0 = jnp.sum(oh0 * slot, axis=1, keepdims=True)
    pos1 = jnp.sum(oh1 * slot, axis=1, keepdims=True)

    nact = jnp.sum(jnp.where(lane[:1, :] == E - 1, pnb, 0.0),
                   axis=1, keepdims=True)
    bf = lane[:1, :].astype(jnp.float32)
    be = jnp.zeros((1, LANES), jnp.float32)
    for e in range(E):
        pnb_e = jnp.sum(jnp.where(lane[:1, :] == e, pnb, 0.0),
                        axis=1, keepdims=True)
        be = be + (bf >= pnb_e).astype(jnp.float32)
    be = jnp.minimum(be, float(E - 1))

    be_prev = jnp.concatenate([be[:, :1] - 1.0, be[:, :NBMAX - 1],
                               jnp.zeros((1, LANES - NBMAX), jnp.float32)],
                              axis=1)
    sta = jnp.where(lane[:1, :] < NBMAX,
                    (be != be_prev).astype(jnp.float32), 0.0)
    rid = sta
    s = 1
    while s < LANES:
        rid = rid + jnp.concatenate(
            [jnp.zeros((1, s), jnp.float32), rid[:, :-s]], axis=1)
        s *= 2
    rid = rid - 1.0
    rpf = jnp.sum(jnp.where(lane[:1, :] == NBMAX - 1, rid + 1.0, 0.0),
                  axis=1, keepdims=True)
    rex = jnp.zeros((1, LANES), jnp.float32)
    for r in range(E):
        rex_r = jnp.min(jnp.where((rid == r) & (lane[:1, :] < NBMAX),
                                  be, float(E)), axis=1, keepdims=True)
        rex = rex + (lane[:1, :] == r).astype(jnp.float32) * rex_r
    rex = jnp.minimum(rex, float(E - 1))

    rowoff = jnp.zeros((1, LANES), jnp.float32)
    for e in range(E):
        off8_e = jnp.sum(jnp.where(lane[:1, :] == e, off8, 0.0),
                         axis=1, keepdims=True)
        pext_e = jnp.sum(jnp.where(lane[:1, :] == e, pext, 0.0),
                         axis=1, keepdims=True)
        rowoff = rowoff + (be == e).astype(jnp.float32) * (
            off8_e + (bf - pext_e) * BT)
    rowoff = jnp.clip(rowoff, 0.0, float(PX - BT))

    meta = jnp.concatenate([be, rex, rid, sta, rowoff,
                            jnp.broadcast_to(nact, (1, LANES)),
                            jnp.broadcast_to(rpf, (1, LANES))], axis=0)

    pos0_ref[...] = pos0.astype(jnp.int32)
    pos1_ref[...] = pos1.astype(jnp.int32)
    w0_ref[...] = jnp.broadcast_to(w0, (T, 16))
    w1_ref[...] = jnp.broadcast_to(w1, (T, 16))
    meta_ref[...] = meta.astype(jnp.int32)


def _routing(x, gw_pad):
    return pl.pallas_call(
        _routing_body,
        out_shape=[
            jax.ShapeDtypeStruct((T, 1), jnp.int32),
            jax.ShapeDtypeStruct((T, 1), jnp.int32),
            jax.ShapeDtypeStruct((T, 16), jnp.float32),
            jax.ShapeDtypeStruct((T, 16), jnp.float32),
            jax.ShapeDtypeStruct((7, LANES), jnp.int32),
        ],
    )(x, gw_pad)


def _dispatch_body(x_hbm, p0_hbm, p1_hbm, out_hbm, idx0_v, idx1_v, rows_v, sem):
    c = lax.axis_index("c")
    s = lax.axis_index("s")
    wid = s * 2 + c
    pltpu.sync_copy(p0_hbm.at[wid], idx0_v)
    pltpu.sync_copy(p1_hbm.at[wid], idx1_v)
    pltpu.sync_copy(x_hbm.at[pl.ds(wid * TPT, TPT)], rows_v)
    pltpu.async_copy(rows_v, out_hbm.at[idx0_v], sem).wait()
    pltpu.async_copy(rows_v, out_hbm.at[idx1_v], sem).wait()


def _dispatch(x, p0, p1):
    mesh = plsc.VectorSubcoreMesh(core_axis_name="c", subcore_axis_name="s")
    fn = functools.partial(
        pl.kernel,
        out_type=jax.ShapeDtypeStruct((PX, D), jnp.float32),
        mesh=mesh,
        scratch_types=[
            pltpu.VMEM((TPT,), jnp.int32),
            pltpu.VMEM((TPT,), jnp.int32),
            pltpu.VMEM((TPT, D), jnp.float32),
            pltpu.SemaphoreType.DMA,
        ],
    )(_dispatch_body)
    return fn(x, p0, p1)


NSLOT = 2


def _gmm_body(bex_ref, rex_ref, rid_ref, sta_ref, roff_ref, nact_ref, rpf_ref,
              xs_hbm, w1_hbm, w3_hbm, w2_hbm, out_ref,
              acc_ref, xs_v, wb1, wb3, wb2, s1, s3, s2, xsem, osem):
    f = pl.program_id(0)
    j = pl.program_id(1)

    @pl.when((f == 0) & (j == 0))
    def _():
        pltpu.make_async_copy(xs_hbm, xs_v, xsem).start()
    ecur = bex_ref[j]
    r = rid_ref[j]
    rpf = rpf_ref[0]
    g = f * rpf + r
    slot = lax.rem(g, NSLOT)

    def copies(e, fidx, sl):
        return (
            pltpu.make_async_copy(
                w1_hbm.at[e, pl.ds(fidx * FFB, FFB)], wb1.at[sl], s1.at[sl]),
            pltpu.make_async_copy(
                w3_hbm.at[e, pl.ds(fidx * FFB, FFB)], wb3.at[sl], s3.at[sl]),
            pltpu.make_async_copy(
                w2_hbm.at[e, :, pl.ds(fidx * FFB, FFB)], wb2.at[sl], s2.at[sl]),
        )

    def issue_run(gt):
        f_t = lax.div(gt, rpf)
        r_t = gt - f_t * rpf

        @pl.when(f_t < NF)
        def _():
            for cp in copies(rex_ref[r_t], f_t, lax.rem(gt, NSLOT)):
                cp.start()

    @pl.when(sta_ref[j] == 1)
    def _():
        @pl.when((f == 0) & (j == 0))
        def _():
            for k in range(NSLOT - 1):
                issue_run(k)
            pltpu.make_async_copy(xs_hbm, xs_v, xsem).wait()
        for cp in copies(ecur, f, slot):
            cp.wait()
        issue_run(g + NSLOT - 1)

    @pl.when(j < nact_ref[0])
    def _():
        roff = pl.multiple_of(roff_ref[j], 8)
        x = xs_v[pl.ds(roff, BT)]
        w1c = wb1[slot]
        w3c = wb3[slot]
        w2c = wb2[slot]
        h1 = lax.dot_general(x, w1c, (((1,), (1,)), ((), ())),
                             preferred_element_type=jnp.float32)
        h3 = lax.dot_general(x, w3c, (((1,), (1,)), ((), ())),
                             preferred_element_type=jnp.float32)
        h = h1 * lax.logistic(h1) * h3
        y = lax.dot_general(h, w2c, (((1,), (1,)), ((), ())),
                            preferred_element_type=jnp.float32)

        @pl.when(f == 0)
        def _():
            acc_ref[pl.ds(j, 1)] = y[None]

        @pl.when(f > 0)
        def _():
            acc_ref[pl.ds(j, 1)] += y[None]

        @pl.when(f == NF - 1)
        def _():
            cp = pltpu.make_async_copy(
                acc_ref.at[j], out_ref.at[pl.ds(roff, BT)], osem)
            cp.start()
            cp.wait()


def _gmm(bex, rex, rid, sta, roff, nact, rpf, xs, w1, w3, w2):
    grid_spec = pltpu.PrefetchScalarGridSpec(
        num_scalar_prefetch=7,
        grid=(NF, NBMAX),
        in_specs=[
            pl.BlockSpec(memory_space=pl.ANY),
            pl.BlockSpec(memory_space=pl.ANY),
            pl.BlockSpec(memory_space=pl.ANY),
            pl.BlockSpec(memory_space=pl.ANY),
        ],
        out_specs=pl.BlockSpec(memory_space=pl.ANY),
        scratch_shapes=[
            pltpu.VMEM((NBMAX, BT, D), jnp.float32),
            pltpu.VMEM((PX, D), jnp.float32),
            pltpu.VMEM((NSLOT, FFB, D), jnp.float32),
            pltpu.VMEM((NSLOT, FFB, D), jnp.float32),
            pltpu.VMEM((NSLOT, D, FFB), jnp.float32),
            pltpu.SemaphoreType.DMA((NSLOT,)),
            pltpu.SemaphoreType.DMA((NSLOT,)),
            pltpu.SemaphoreType.DMA((NSLOT,)),
            pltpu.SemaphoreType.DMA,
            pltpu.SemaphoreType.DMA,
        ],
    )
    return pl.pallas_call(
        _gmm_body,
        grid_spec=grid_spec,
        out_shape=jax.ShapeDtypeStruct((PX, D), jnp.float32),
        compiler_params=pltpu.CompilerParams(
            dimension_semantics=("arbitrary", "arbitrary")),
    )(bex, rex, rid, sta, roff, nact, rpf, xs, w1, w3, w2)


def _combine_body(ys_hbm, p0_hbm, p1_hbm, w0_hbm, w1_hbm, out_hbm,
                  idx0_v, idx1_v, w0_v, w1_v, g0_v, g1_v, sem0, sem1):
    c = lax.axis_index("c")
    s = lax.axis_index("s")
    wid = s * 2 + c
    pltpu.sync_copy(p0_hbm.at[wid], idx0_v)
    pltpu.sync_copy(p1_hbm.at[wid], idx1_v)
    pltpu.sync_copy(w0_hbm.at[wid], w0_v)
    pltpu.sync_copy(w1_hbm.at[wid], w1_v)
    half_n = TPT // 2
    for half in range(2):
        cp0 = pltpu.async_copy(
            ys_hbm.at[idx0_v.at[pl.ds(half * half_n, half_n)]], g0_v, sem0)
        cp1 = pltpu.async_copy(
            ys_hbm.at[idx1_v.at[pl.ds(half * half_n, half_n)]], g1_v, sem1)
        cp0.wait()
        cp1.wait()

        def row_body(r, _, half=half):
            a = w0_v[half * half_n + r, :]
            bw = w1_v[half * half_n + r, :]
            for cc in range(D // 16):
                sl = pl.ds(cc * 16, 16)
                g0_v[r, sl] = a * g0_v[r, sl] + bw * g1_v[r, sl]
            return 0

        lax.fori_loop(0, half_n, row_body, 0)
        pltpu.sync_copy(g0_v, out_hbm.at[pl.ds(wid * TPT + half * half_n, half_n)])


def _combine(ys, p0, p1, w0m, w1m):
    mesh = plsc.VectorSubcoreMesh(core_axis_name="c", subcore_axis_name="s")
    half_n = TPT // 2
    fn = functools.partial(
        pl.kernel,
        out_type=jax.ShapeDtypeStruct((T, D), jnp.float32),
        mesh=mesh,
        scratch_types=[
            pltpu.VMEM((TPT,), jnp.int32),
            pltpu.VMEM((TPT,), jnp.int32),
            pltpu.VMEM((TPT, 16), jnp.float32),
            pltpu.VMEM((TPT, 16), jnp.float32),
            pltpu.VMEM((half_n, D), jnp.float32),
            pltpu.VMEM((half_n, D), jnp.float32),
            pltpu.SemaphoreType.DMA,
            pltpu.SemaphoreType.DMA,
        ],
    )(_combine_body)
    return fn(ys, p0, p1, w0m, w1m)


def kernel(hidden_states, gate_w, w1, w2, w3):
    b, s, d = hidden_states.shape
    x = hidden_states.reshape(-1, d)
    gw_pad = jnp.pad(gate_w, ((0, LANES - E), (0, 0)))
    pos0, pos1, w0c, w1c, meta = _routing(x, gw_pad)
    p0 = pos0.reshape(NTILES, TPT)
    p1 = pos1.reshape(NTILES, TPT)
    w0m = w0c.reshape(NTILES, TPT, 16)
    w1m = w1c.reshape(NTILES, TPT, 16)
    bex = meta[0, :NBMAX]
    rex = meta[1, :E]
    rid = meta[2, :NBMAX]
    sta = meta[3, :NBMAX]
    roff = meta[4, :NBMAX]
    nact = meta[5, :1]
    rpf = meta[6, :1]
    xs = _dispatch(x, p0, p1)
    ys = _gmm(bex, rex, rid, sta, roff, nact, rpf, xs, w1, w3, w2)
    out = _combine(ys, p0, p1, w0m, w1m)
    return out.reshape(b, s, d)


# speedup vs baseline: 1.7354x; 1.0038x over previous
import functools

import jax
import jax.numpy as jnp
from jax import lax
from jax.experimental import pallas as pl
from jax.experimental.pallas import tpu as pltpu
from jax.experimental.pallas import tpu_sc as plsc

E = 8
TOPK = 2
T = 2048
D = 1024
FF = 3584

BT = 256
NBMAX = -((-2 * T) // BT) + E - 1
P8 = 2 * T + 8 * (E - 1)
PX = P8 + BT
FFB = 512
NF = FF // FFB

NTILES = 32
TPT = T // NTILES
LANES = 128


def _routing_body(x_ref, gw_ref, pos0_ref, pos1_ref, w0_ref, w1_ref, meta_ref):
    x = x_ref[...]
    gw = gw_ref[...]
    logits = lax.dot_general(x, gw, (((1,), (1,)), ((), ())),
                             preferred_element_type=jnp.float32)
    lane = lax.broadcasted_iota(jnp.int32, (T, LANES), 1)
    neg = jnp.float32(-1e30)
    logits = jnp.where(lane < E, logits, neg)

    m0 = jnp.max(logits, axis=1, keepdims=True)
    i0 = jnp.min(jnp.where(logits == m0, lane, LANES), axis=1, keepdims=True)
    l2 = jnp.where(lane == i0, neg, logits)
    m1 = jnp.max(l2, axis=1, keepdims=True)
    i1 = jnp.min(jnp.where(l2 == m1, lane, LANES), axis=1, keepdims=True)

    ex = jnp.exp(m1 - m0)
    w0 = 1.0 / (1.0 + ex)
    w1 = ex / (1.0 + ex)

    oh0 = (lane == i0).astype(jnp.float32)
    oh1 = (lane == i1).astype(jnp.float32)
    cnt = oh0 + oh1

    csum = cnt
    s = 1
    while s < T:
        csum = csum + jnp.concatenate(
            [jnp.zeros((s, LANES), jnp.float32), csum[:-s, :]], axis=0)
        s *= 2
    excl = csum - cnt
    counts = csum[T - 1:T, :]

    nb = jnp.floor((counts + (BT - 1)) / BT)
    nb = jnp.where(lane[:1, :] < E, nb, 0.0)
    pnb = nb
    s = 1
    while s < LANES:
        pnb = pnb + jnp.concatenate(
            [jnp.zeros((1, s), jnp.float32), pnb[:, :-s]], axis=1)
        s *= 2
    pext = pnb - nb

    r8 = 8.0 * jnp.floor((counts + 7.0) / 8.0)
    r8 = jnp.where(lane[:1, :] < E, r8, 0.0)
    po8 = r8
    s = 1
    while s < LANES:
        po8 = po8 + jnp.concatenate(
            [jnp.zeros((1, s), jnp.float32), po8[:, :-s]], axis=1)
        s *= 2
    off8 = po8 - r8

    slot = excl + off8
    pos0 = jnp.sum(oh0 * slot, axis=1, keepdims=True)
    pos1 = jnp.sum(oh1 * slot, axis=1, keepdims=True)

    nact = jnp.sum(jnp.where(lane[:1, :] == E - 1, pnb, 0.0),
                   axis=1, keepdims=True)
    bf = lane[:1, :].astype(jnp.float32)
    be = jnp.zeros((1, LANES), jnp.float32)
    for e in range(E):
        pnb_e = jnp.sum(jnp.where(lane[:1, :] == e, pnb, 0.0),
                        axis=1, keepdims=True)
        be = be + (bf >= pnb_e).astype(jnp.float32)
    be = jnp.minimum(be, float(E - 1))

    be_prev = jnp.concatenate([be[:, :1] - 1.0, be[:, :NBMAX - 1],
                               jnp.zeros((1, LANES - NBMAX), jnp.float32)],
                              axis=1)
    sta = jnp.where(lane[:1, :] < NBMAX,
                    (be != be_prev).astype(jnp.float32), 0.0)
    rid = sta
    s = 1
    while s < LANES:
        rid = rid + jnp.concatenate(
            [jnp.zeros((1, s), jnp.float32), rid[:, :-s]], axis=1)
        s *= 2
    rid = rid - 1.0
    rpf = jnp.sum(jnp.where(lane[:1, :] == NBMAX - 1, rid + 1.0, 0.0),
                  axis=1, keepdims=True)
    rex = jnp.zeros((1, LANES), jnp.float32)
    for r in range(E):
        rex_r = jnp.min(jnp.where((rid == r) & (lane[:1, :] < NBMAX),
                                  be, float(E)), axis=1, keepdims=True)
        rex = rex + (lane[:1, :] == r).astype(jnp.float32) * rex_r
    rex = jnp.minimum(rex, float(E - 1))

    rowoff = jnp.zeros((1, LANES), jnp.float32)
    for e in range(E):
        off8_e = jnp.sum(jnp.where(lane[:1, :] == e, off8, 0.0),
                         axis=1, keepdims=True)
        pext_e = jnp.sum(jnp.where(lane[:1, :] == e, pext, 0.0),
                         axis=1, keepdims=True)
        rowoff = rowoff + (be == e).astype(jnp.float32) * (
            off8_e + (bf - pext_e) * BT)
    rowoff = jnp.clip(rowoff, 0.0, float(PX - BT))

    meta = jnp.concatenate([be, rex, rid, sta, rowoff,
                            jnp.broadcast_to(nact, (1, LANES)),
                            jnp.broadcast_to(rpf, (1, LANES))], axis=0)

    pos0_ref[...] = pos0.astype(jnp.int32)
    pos1_ref[...] = pos1.astype(jnp.int32)
    w0_ref[...] = jnp.broadcast_to(w0, (T, 16))
    w1_ref[...] = jnp.broadcast_to(w1, (T, 16))
    meta_ref[...] = meta.astype(jnp.int32)


def _routing(x, gw_pad):
    return pl.pallas_call(
        _routing_body,
        out_shape=[
            jax.ShapeDtypeStruct((T, 1), jnp.int32),
            jax.ShapeDtypeStruct((T, 1), jnp.int32),
            jax.ShapeDtypeStruct((T, 16), jnp.float32),
            jax.ShapeDtypeStruct((T, 16), jnp.float32),
            jax.ShapeDtypeStruct((7, LANES), jnp.int32),
        ],
    )(x, gw_pad)


def _dispatch_body(x_hbm, p0_hbm, p1_hbm, out_hbm, idx0_v, idx1_v, rows_v, sem):
    c = lax.axis_index("c")
    s = lax.axis_index("s")
    wid = s * 2 + c
    pltpu.sync_copy(p0_hbm.at[wid], idx0_v)
    pltpu.sync_copy(p1_hbm.at[wid], idx1_v)
    pltpu.sync_copy(x_hbm.at[pl.ds(wid * TPT, TPT)], rows_v)
    pltpu.async_copy(rows_v, out_hbm.at[idx0_v], sem).wait()
    pltpu.async_copy(rows_v, out_hbm.at[idx1_v], sem).wait()


def _dispatch(x, p0, p1):
    mesh = plsc.VectorSubcoreMesh(core_axis_name="c", subcore_axis_name="s")
    fn = functools.partial(
        pl.kernel,
        out_type=jax.ShapeDtypeStruct((PX, D), jnp.float32),
        mesh=mesh,
        scratch_types=[
            pltpu.VMEM((TPT,), jnp.int32),
            pltpu.VMEM((TPT,), jnp.int32),
            pltpu.VMEM((TPT, D), jnp.float32),
            pltpu.SemaphoreType.DMA,
        ],
    )(_dispatch_body)
    return fn(x, p0, p1)


NSLOT = 2


def _gmm_body(bex_ref, rex_ref, rid_ref, sta_ref, roff_ref, nact_ref, rpf_ref,
              xs_hbm, w1_hbm, w3_hbm, w2_hbm, out_ref,
              acc_ref, xs_v, wb1, wb3, wb2, s1, s3, s2, xsem, osem):
    f = pl.program_id(0)
    j = pl.program_id(1)

    @pl.when((f == 0) & (j == 0))
    def _():
        pltpu.make_async_copy(xs_hbm, xs_v, xsem).start()
    ecur = bex_ref[j]
    r = rid_ref[j]
    rpf = rpf_ref[0]
    g = f * rpf + r
    slot = lax.rem(g, NSLOT)

    def copies(e, fidx, sl):
        return (
            pltpu.make_async_copy(
                w1_hbm.at[e, pl.ds(fidx * FFB, FFB)], wb1.at[sl], s1.at[sl]),
            pltpu.make_async_copy(
                w3_hbm.at[e, pl.ds(fidx * FFB, FFB)], wb3.at[sl], s3.at[sl]),
            pltpu.make_async_copy(
                w2_hbm.at[e, :, pl.ds(fidx * FFB, FFB)], wb2.at[sl], s2.at[sl]),
        )

    def issue_run(gt):
        f_t = lax.div(gt, rpf)
        r_t = gt - f_t * rpf

        @pl.when(f_t < NF)
        def _():
            for cp in copies(rex_ref[r_t], f_t, lax.rem(gt, NSLOT)):
                cp.start()

    @pl.when(sta_ref[j] == 1)
    def _():
        @pl.when((f == 0) & (j == 0))
        def _():
            for k in range(NSLOT - 1):
                issue_run(k)
            pltpu.make_async_copy(xs_hbm, xs_v, xsem).wait()
        for cp in copies(ecur, f, slot):
            cp.wait()
        issue_run(g + NSLOT - 1)

    @pl.when(j < nact_ref[0])
    def _():
        roff = pl.multiple_of(roff_ref[j], 8)
        x = xs_v[pl.ds(roff, BT)]
        w1c = wb1[slot]
        w3c = wb3[slot]
        w2c = wb2[slot]
        h1 = lax.dot_general(x, w1c, (((1,), (1,)), ((), ())),
                             preferred_element_type=jnp.float32)
        h3 = lax.dot_general(x, w3c, (((1,), (1,)), ((), ())),
                             preferred_element_type=jnp.float32)
        h = h1 * lax.logistic(h1) * h3
        y = lax.dot_general(h, w2c, (((1,), (1,)), ((), ())),
                            preferred_element_type=jnp.float32)

        @pl.when(f == 0)
        def _():
            acc_ref[pl.ds(j, 1)] = y[None]

        @pl.when(f > 0)
        def _():
            acc_ref[pl.ds(j, 1)] += y[None]

        @pl.when(f == NF - 1)
        def _():
            cp = pltpu.make_async_copy(
                acc_ref.at[j], out_ref.at[pl.ds(roff, BT)], osem)
            cp.start()
            cp.wait()


def _gmm(bex, rex, rid, sta, roff, nact, rpf, xs, w1, w3, w2):
    grid_spec = pltpu.PrefetchScalarGridSpec(
        num_scalar_prefetch=7,
        grid=(NF, NBMAX),
        in_specs=[
            pl.BlockSpec(memory_space=pl.ANY),
            pl.BlockSpec(memory_space=pl.ANY),
            pl.BlockSpec(memory_space=pl.ANY),
            pl.BlockSpec(memory_space=pl.ANY),
        ],
        out_specs=pl.BlockSpec(memory_space=pl.ANY),
        scratch_shapes=[
            pltpu.VMEM((NBMAX, BT, D), jnp.float32),
            pltpu.VMEM((PX, D), jnp.float32),
            pltpu.VMEM((NSLOT, FFB, D), jnp.float32),
            pltpu.VMEM((NSLOT, FFB, D), jnp.float32),
            pltpu.VMEM((NSLOT, D, FFB), jnp.float32),
            pltpu.SemaphoreType.DMA((NSLOT,)),
            pltpu.SemaphoreType.DMA((NSLOT,)),
            pltpu.SemaphoreType.DMA((NSLOT,)),
            pltpu.SemaphoreType.DMA,
            pltpu.SemaphoreType.DMA,
        ],
    )
    return pl.pallas_call(
        _gmm_body,
        grid_spec=grid_spec,
        out_shape=jax.ShapeDtypeStruct((PX, D), jnp.float32),
        compiler_params=pltpu.CompilerParams(
            dimension_semantics=("arbitrary", "arbitrary")),
    )(bex, rex, rid, sta, roff, nact, rpf, xs, w1, w3, w2)


def _combine_body(ys_hbm, p0_hbm, p1_hbm, w0_hbm, w1_hbm, out_hbm,
                  idx0_v, idx1_v, w0_v, w1_v, g0_v, g1_v, sem0, sem1):
    c = lax.axis_index("c")
    s = lax.axis_index("s")
    wid = s * 2 + c
    pltpu.sync_copy(p0_hbm.at[wid], idx0_v)
    pltpu.sync_copy(p1_hbm.at[wid], idx1_v)
    pltpu.sync_copy(w0_hbm.at[wid], w0_v)
    pltpu.sync_copy(w1_hbm.at[wid], w1_v)
    half_n = TPT // 2
    for half in range(2):
        cp0 = pltpu.async_copy(
            ys_hbm.at[idx0_v.at[pl.ds(half * half_n, half_n)]], g0_v, sem0)
        cp1 = pltpu.async_copy(
            ys_hbm.at[idx1_v.at[pl.ds(half * half_n, half_n)]], g1_v, sem1)
        cp0.wait()
        cp1.wait()

        def row_body(r, _, half=half):
            a = w0_v[half * half_n + r, :]
            bw = w1_v[half * half_n + r, :]
            for cc in range(D // 16):
                sl = pl.ds(cc * 16, 16)
                g0_v[r, sl] = a * g0_v[r, sl] + bw * g1_v[r, sl]
            return 0

        lax.fori_loop(0, half_n, row_body, 0)
        pltpu.sync_copy(g0_v, out_hbm.at[pl.ds(wid * TPT + half * half_n, half_n)])


def _combine(ys, p0, p1, w0m, w1m):
    mesh = plsc.VectorSubcoreMesh(core_axis_name="c", subcore_axis_name="s")
    half_n = TPT // 2
    fn = functools.partial(
        pl.kernel,
        out_type=jax.ShapeDtypeStruct((T, D), jnp.float32),
        mesh=mesh,
        scratch_types=[
            pltpu.VMEM((TPT,), jnp.int32),
            pltpu.VMEM((TPT,), jnp.int32),
            pltpu.VMEM((TPT, 16), jnp.float32),
            pltpu.VMEM((TPT, 16), jnp.float32),
            pltpu.VMEM((half_n, D), jnp.float32),
            pltpu.VMEM((half_n, D), jnp.float32),
            pltpu.SemaphoreType.DMA,
            pltpu.SemaphoreType.DMA,
        ],
    )(_combine_body)
    return fn(ys, p0, p1, w0m, w1m)


def kernel(hidden_states, gate_w, w1, w2, w3):
    b, s, d = hidden_states.shape
    x = hidden_states.reshape(-1, d)
    gw_pad = jnp.pad(gate_w, ((0, LANES - E), (0, 0)))
    pos0, pos1, w0c, w1c, meta = _routing(x, gw_pad)
    p0 = pos0.reshape(NTILES, TPT)
    p1 = pos1.reshape(NTILES, TPT)
    w0m = w0c.reshape(NTILES, TPT, 16)
    w1m = w1c.reshape(NTILES, TPT, 16)
    bex = meta[0, :NBMAX]
    rex = meta[1, :E]
    rid = meta[2, :NBMAX]
    sta = meta[3, :NBMAX]
    roff = meta[4, :NBMAX]
    nact = meta[5, :1]
    rpf = meta[6, :1]
    xs = _dispatch(x, p0, p1)
    ys = _gmm(bex, rex, rid, sta, roff, nact, rpf, xs, w1, w3, w2)
    out = _combine(ys, p0, p1, w0m, w1m)
    return out.reshape(b, s, d)


# speedup vs baseline: 1.8233x; 1.0506x over previous
import functools

import jax
import jax.numpy as jnp
from jax import lax
from jax.experimental import pallas as pl
from jax.experimental.pallas import tpu as pltpu
from jax.experimental.pallas import tpu_sc as plsc

E = 8
TOPK = 2
T = 2048
D = 1024
FF = 3584

BT = 256
NBMAX = -((-2 * T) // BT) + E - 1
P8 = 2 * T + 8 * (E - 1)
PX = P8 + BT
FFB = 512
NF = FF // FFB

NTILES = 32
TPT = T // NTILES
LANES = 128


def _routing_body(x_ref, gw_ref, pos0_ref, pos1_ref, w0_ref, w1_ref, meta_ref):
    x = x_ref[...]
    gw = gw_ref[...]
    logits = lax.dot_general(x, gw, (((1,), (1,)), ((), ())),
                             preferred_element_type=jnp.float32)
    lane = lax.broadcasted_iota(jnp.int32, (T, LANES), 1)
    neg = jnp.float32(-1e30)
    logits = jnp.where(lane < E, logits, neg)

    m0 = jnp.max(logits, axis=1, keepdims=True)
    i0 = jnp.min(jnp.where(logits == m0, lane, LANES), axis=1, keepdims=True)
    l2 = jnp.where(lane == i0, neg, logits)
    m1 = jnp.max(l2, axis=1, keepdims=True)
    i1 = jnp.min(jnp.where(l2 == m1, lane, LANES), axis=1, keepdims=True)

    ex = jnp.exp(m1 - m0)
    w0 = 1.0 / (1.0 + ex)
    w1 = ex / (1.0 + ex)

    oh0 = (lane == i0).astype(jnp.float32)
    oh1 = (lane == i1).astype(jnp.float32)
    cnt = oh0 + oh1

    csum = cnt
    s = 1
    while s < T:
        csum = csum + jnp.concatenate(
            [jnp.zeros((s, LANES), jnp.float32), csum[:-s, :]], axis=0)
        s *= 2
    excl = csum - cnt
    counts = csum[T - 1:T, :]

    nb = jnp.floor((counts + (BT - 1)) / BT)
    nb = jnp.where(lane[:1, :] < E, nb, 0.0)
    pnb = nb
    s = 1
    while s < LANES:
        pnb = pnb + jnp.concatenate(
            [jnp.zeros((1, s), jnp.float32), pnb[:, :-s]], axis=1)
        s *= 2
    pext = pnb - nb

    r8 = 8.0 * jnp.floor((counts + 7.0) / 8.0)
    r8 = jnp.where(lane[:1, :] < E, r8, 0.0)
    po8 = r8
    s = 1
    while s < LANES:
        po8 = po8 + jnp.concatenate(
            [jnp.zeros((1, s), jnp.float32), po8[:, :-s]], axis=1)
        s *= 2
    off8 = po8 - r8

    slot = excl + off8
    pos0 = jnp.sum(oh0 * slot, axis=1, keepdims=True)
    pos1 = jnp.sum(oh1 * slot, axis=1, keepdims=True)

    nact = jnp.sum(jnp.where(lane[:1, :] == E - 1, pnb, 0.0),
                   axis=1, keepdims=True)
    bf = lane[:1, :].astype(jnp.float32)
    be = jnp.zeros((1, LANES), jnp.float32)
    for e in range(E):
        pnb_e = jnp.sum(jnp.where(lane[:1, :] == e, pnb, 0.0),
                        axis=1, keepdims=True)
        be = be + (bf >= pnb_e).astype(jnp.float32)
    be = jnp.minimum(be, float(E - 1))

    be_prev = jnp.concatenate([be[:, :1] - 1.0, be[:, :NBMAX - 1],
                               jnp.zeros((1, LANES - NBMAX), jnp.float32)],
                              axis=1)
    sta = jnp.where(lane[:1, :] < NBMAX,
                    (be != be_prev).astype(jnp.float32), 0.0)
    rid = sta
    s = 1
    while s < LANES:
        rid = rid + jnp.concatenate(
            [jnp.zeros((1, s), jnp.float32), rid[:, :-s]], axis=1)
        s *= 2
    rid = rid - 1.0
    rpf = jnp.sum(jnp.where(lane[:1, :] == NBMAX - 1, rid + 1.0, 0.0),
                  axis=1, keepdims=True)
    rex = jnp.zeros((1, LANES), jnp.float32)
    for r in range(E):
        rex_r = jnp.min(jnp.where((rid == r) & (lane[:1, :] < NBMAX),
                                  be, float(E)), axis=1, keepdims=True)
        rex = rex + (lane[:1, :] == r).astype(jnp.float32) * rex_r
    rex = jnp.minimum(rex, float(E - 1))

    rowoff = jnp.zeros((1, LANES), jnp.float32)
    for e in range(E):
        off8_e = jnp.sum(jnp.where(lane[:1, :] == e, off8, 0.0),
                         axis=1, keepdims=True)
        pext_e = jnp.sum(jnp.where(lane[:1, :] == e, pext, 0.0),
                         axis=1, keepdims=True)
        rowoff = rowoff + (be == e).astype(jnp.float32) * (
            off8_e + (bf - pext_e) * BT)
    rowoff = jnp.clip(rowoff, 0.0, float(PX - BT))

    meta = jnp.concatenate([be, rex, rid, sta, rowoff,
                            jnp.broadcast_to(nact, (1, LANES)),
                            jnp.broadcast_to(rpf, (1, LANES))], axis=0)

    pos0_ref[...] = pos0.astype(jnp.int32)
    pos1_ref[...] = pos1.astype(jnp.int32)
    w0_ref[...] = jnp.broadcast_to(w0, (T, 16))
    w1_ref[...] = jnp.broadcast_to(w1, (T, 16))
    meta_ref[...] = meta.astype(jnp.int32)


def _routing(x, gw_pad):
    return pl.pallas_call(
        _routing_body,
        out_shape=[
            jax.ShapeDtypeStruct((T, 1), jnp.int32),
            jax.ShapeDtypeStruct((T, 1), jnp.int32),
            jax.ShapeDtypeStruct((T, 16), jnp.float32),
            jax.ShapeDtypeStruct((T, 16), jnp.float32),
            jax.ShapeDtypeStruct((7, LANES), jnp.int32),
        ],
    )(x, gw_pad)


def _dispatch_body(x_hbm, p0_hbm, p1_hbm, out_hbm, idx0_v, idx1_v, rows_v, sem):
    c = lax.axis_index("c")
    s = lax.axis_index("s")
    wid = s * 2 + c
    pltpu.sync_copy(p0_hbm.at[wid], idx0_v)
    pltpu.sync_copy(p1_hbm.at[wid], idx1_v)
    pltpu.sync_copy(x_hbm.at[pl.ds(wid * TPT, TPT)], rows_v)
    pltpu.async_copy(rows_v, out_hbm.at[idx0_v], sem).wait()
    pltpu.async_copy(rows_v, out_hbm.at[idx1_v], sem).wait()


def _dispatch(x, p0, p1):
    mesh = plsc.VectorSubcoreMesh(core_axis_name="c", subcore_axis_name="s")
    fn = functools.partial(
        pl.kernel,
        out_type=jax.ShapeDtypeStruct((PX, D), jnp.float32),
        mesh=mesh,
        scratch_types=[
            pltpu.VMEM((TPT,), jnp.int32),
            pltpu.VMEM((TPT,), jnp.int32),
            pltpu.VMEM((TPT, D), jnp.float32),
            pltpu.SemaphoreType.DMA,
        ],
    )(_dispatch_body)
    return fn(x, p0, p1)


NSLOT = 2


def _gmm_body(bex_ref, rex_ref, rid_ref, sta_ref, roff_ref, nact_ref, rpf_ref,
              xs_hbm, w1_hbm, w3_hbm, w2_hbm, out_ref,
              acc_ref, xs_v, wb1, wb3, wb2, s1, s3, s2, xsem, osem):
    f = pl.program_id(0)
    j = pl.program_id(1)

    @pl.when((f == 0) & (j == 0))
    def _():
        pltpu.make_async_copy(xs_hbm, xs_v, xsem).start()
    ecur = bex_ref[j]
    r = rid_ref[j]
    rpf = rpf_ref[0]
    g = f * rpf + r
    slot = lax.rem(g, NSLOT)

    def copies(e, fidx, sl):
        return (
            pltpu.make_async_copy(
                w1_hbm.at[e, pl.ds(fidx * FFB, FFB)], wb1.at[sl], s1.at[sl]),
            pltpu.make_async_copy(
                w3_hbm.at[e, pl.ds(fidx * FFB, FFB)], wb3.at[sl], s3.at[sl]),
            pltpu.make_async_copy(
                w2_hbm.at[e, :, pl.ds(fidx * FFB, FFB)], wb2.at[sl], s2.at[sl]),
        )

    def issue_run(gt):
        f_t = lax.div(gt, rpf)
        r_t = gt - f_t * rpf

        @pl.when(f_t < NF)
        def _():
            for cp in copies(rex_ref[r_t], f_t, lax.rem(gt, NSLOT)):
                cp.start()

    @pl.when(sta_ref[j] == 1)
    def _():
        @pl.when((f == 0) & (j == 0))
        def _():
            for k in range(NSLOT - 1):
                issue_run(k)
            pltpu.make_async_copy(xs_hbm, xs_v, xsem).wait()
        for cp in copies(ecur, f, slot):
            cp.wait()
        issue_run(g + NSLOT - 1)

    @pl.when(j < nact_ref[0])
    def _():
        roff = pl.multiple_of(roff_ref[j], 8)
        x = xs_v[pl.ds(roff, BT)]
        w1c = wb1[slot]
        w3c = wb3[slot]
        w2c = wb2[slot]
        h1 = lax.dot_general(x, w1c, (((1,), (1,)), ((), ())),
                             preferred_element_type=jnp.float32)
        h3 = lax.dot_general(x, w3c, (((1,), (1,)), ((), ())),
                             preferred_element_type=jnp.float32)
        h = h1 * lax.logistic(h1) * h3
        y = lax.dot_general(h, w2c, (((1,), (1,)), ((), ())),
                            preferred_element_type=jnp.float32)

        @pl.when(f == 0)
        def _():
            acc_ref[pl.ds(j, 1)] = y[None]

        @pl.when(f > 0)
        def _():
            acc_ref[pl.ds(j, 1)] += y[None]

        @pl.when(f == NF - 1)
        def _():
            @pl.when(j > 0)
            def _():
                roffp = pl.multiple_of(roff_ref[j - 1], 8)
                pltpu.make_async_copy(
                    acc_ref.at[j - 1], out_ref.at[pl.ds(roffp, BT)], osem).wait()
            pltpu.make_async_copy(
                acc_ref.at[j], out_ref.at[pl.ds(roff, BT)], osem).start()

    @pl.when((f == NF - 1) & (j == NBMAX - 1))
    def _():
        jl = nact_ref[0] - 1
        roffl = pl.multiple_of(roff_ref[jl], 8)
        pltpu.make_async_copy(
            acc_ref.at[jl], out_ref.at[pl.ds(roffl, BT)], osem).wait()


def _gmm(bex, rex, rid, sta, roff, nact, rpf, xs, w1, w3, w2):
    grid_spec = pltpu.PrefetchScalarGridSpec(
        num_scalar_prefetch=7,
        grid=(NF, NBMAX),
        in_specs=[
            pl.BlockSpec(memory_space=pl.ANY),
            pl.BlockSpec(memory_space=pl.ANY),
            pl.BlockSpec(memory_space=pl.ANY),
            pl.BlockSpec(memory_space=pl.ANY),
        ],
        out_specs=pl.BlockSpec(memory_space=pl.ANY),
        scratch_shapes=[
            pltpu.VMEM((NBMAX, BT, D), jnp.float32),
            pltpu.VMEM((PX, D), jnp.float32),
            pltpu.VMEM((NSLOT, FFB, D), jnp.float32),
            pltpu.VMEM((NSLOT, FFB, D), jnp.float32),
            pltpu.VMEM((NSLOT, D, FFB), jnp.float32),
            pltpu.SemaphoreType.DMA((NSLOT,)),
            pltpu.SemaphoreType.DMA((NSLOT,)),
            pltpu.SemaphoreType.DMA((NSLOT,)),
            pltpu.SemaphoreType.DMA,
            pltpu.SemaphoreType.DMA,
        ],
    )
    return pl.pallas_call(
        _gmm_body,
        grid_spec=grid_spec,
        out_shape=jax.ShapeDtypeStruct((PX, D), jnp.float32),
        compiler_params=pltpu.CompilerParams(
            dimension_semantics=("arbitrary", "arbitrary")),
    )(bex, rex, rid, sta, roff, nact, rpf, xs, w1, w3, w2)


def _combine_body(ys_hbm, p0_hbm, p1_hbm, w0_hbm, w1_hbm, out_hbm,
                  idx0_v, idx1_v, w0_v, w1_v, g0a, g1a, g0b, g1b, sem0, sem1):
    c = lax.axis_index("c")
    s = lax.axis_index("s")
    wid = s * 2 + c
    pltpu.sync_copy(p0_hbm.at[wid], idx0_v)
    pltpu.sync_copy(p1_hbm.at[wid], idx1_v)
    pltpu.sync_copy(w0_hbm.at[wid], w0_v)
    pltpu.sync_copy(w1_hbm.at[wid], w1_v)
    qn = TPT // 4
    bufs = ((g0a, g1a), (g0b, g1b))

    def gather(q):
        b0, b1 = bufs[q % 2]
        return (
            pltpu.async_copy(ys_hbm.at[idx0_v.at[pl.ds(q * qn, qn)]], b0, sem0),
            pltpu.async_copy(ys_hbm.at[idx1_v.at[pl.ds(q * qn, qn)]], b1, sem1),
        )

    pend = {0: gather(0)}
    for q in range(4):
        if q + 1 < 4:
            pend[q + 1] = gather(q + 1)
        for cp in pend[q]:
            cp.wait()
        b0, b1 = bufs[q % 2]

        def row_body(rr, _, q=q, b0=b0, b1=b1):
            for r in (2 * rr, 2 * rr + 1):
                a = w0_v[q * qn + r, :]
                bw = w1_v[q * qn + r, :]
                for cc in range(D // 16):
                    sl = pl.ds(cc * 16, 16)
                    b0[r, sl] = a * b0[r, sl] + bw * b1[r, sl]
            return 0

        lax.fori_loop(0, qn // 2, row_body, 0)
        pltpu.sync_copy(b0, out_hbm.at[pl.ds(wid * TPT + q * qn, qn)])


def _combine(ys, p0, p1, w0m, w1m):
    mesh = plsc.VectorSubcoreMesh(core_axis_name="c", subcore_axis_name="s")
    qn = TPT // 4
    fn = functools.partial(
        pl.kernel,
        out_type=jax.ShapeDtypeStruct((T, D), jnp.float32),
        mesh=mesh,
        scratch_types=[
            pltpu.VMEM((TPT,), jnp.int32),
            pltpu.VMEM((TPT,), jnp.int32),
            pltpu.VMEM((TPT, 16), jnp.float32),
            pltpu.VMEM((TPT, 16), jnp.float32),
            pltpu.VMEM((qn, D), jnp.float32),
            pltpu.VMEM((qn, D), jnp.float32),
            pltpu.VMEM((qn, D), jnp.float32),
            pltpu.VMEM((qn, D), jnp.float32),
            pltpu.SemaphoreType.DMA,
            pltpu.SemaphoreType.DMA,
        ],
    )(_combine_body)
    return fn(ys, p0, p1, w0m, w1m)


def kernel(hidden_states, gate_w, w1, w2, w3):
    b, s, d = hidden_states.shape
    x = hidden_states.reshape(-1, d)
    gw_pad = jnp.pad(gate_w, ((0, LANES - E), (0, 0)))
    pos0, pos1, w0c, w1c, meta = _routing(x, gw_pad)
    p0 = pos0.reshape(NTILES, TPT)
    p1 = pos1.reshape(NTILES, TPT)
    w0m = w0c.reshape(NTILES, TPT, 16)
    w1m = w1c.reshape(NTILES, TPT, 16)
    bex = meta[0, :NBMAX]
    rex = meta[1, :E]
    rid = meta[2, :NBMAX]
    sta = meta[3, :NBMAX]
    roff = meta[4, :NBMAX]
    nact = meta[5, :1]
    rpf = meta[6, :1]
    xs = _dispatch(x, p0, p1)
    ys = _gmm(bex, rex, rid, sta, roff, nact, rpf, xs, w1, w3, w2)
    out = _combine(ys, p0, p1, w0m, w1m)
    return out.reshape(b, s, d)


# speedup vs baseline: 1.8293x; 1.0033x over previous
import functools

import jax
import jax.numpy as jnp
from jax import lax
from jax.experimental import pallas as pl
from jax.experimental.pallas import tpu as pltpu
from jax.experimental.pallas import tpu_sc as plsc

E = 8
TOPK = 2
T = 2048
D = 1024
FF = 3584

BT = 256
NBMAX = -((-2 * T) // BT) + E - 1
P8 = 2 * T + 8 * (E - 1)
PX = P8 + BT
FFB = 512
NF = FF // FFB

NTILES = 32
TPT = T // NTILES
LANES = 128


def _routing_body(x_ref, gw_ref, pos0_ref, pos1_ref, w0_ref, w1_ref, meta_ref):
    x = x_ref[...]
    gw = gw_ref[...]
    logits = lax.dot_general(x, gw, (((1,), (1,)), ((), ())),
                             preferred_element_type=jnp.float32)
    lane = lax.broadcasted_iota(jnp.int32, (T, LANES), 1)
    neg = jnp.float32(-1e30)
    logits = jnp.where(lane < E, logits, neg)

    m0 = jnp.max(logits, axis=1, keepdims=True)
    i0 = jnp.min(jnp.where(logits == m0, lane, LANES), axis=1, keepdims=True)
    l2 = jnp.where(lane == i0, neg, logits)
    m1 = jnp.max(l2, axis=1, keepdims=True)
    i1 = jnp.min(jnp.where(l2 == m1, lane, LANES), axis=1, keepdims=True)

    ex = jnp.exp(m1 - m0)
    w0 = 1.0 / (1.0 + ex)
    w1 = ex / (1.0 + ex)

    oh0 = (lane == i0).astype(jnp.float32)
    oh1 = (lane == i1).astype(jnp.float32)
    cnt = oh0 + oh1

    csum = cnt
    s = 1
    while s < T:
        csum = csum + jnp.concatenate(
            [jnp.zeros((s, LANES), jnp.float32), csum[:-s, :]], axis=0)
        s *= 2
    excl = csum - cnt
    counts = csum[T - 1:T, :]

    nb = jnp.floor((counts + (BT - 1)) / BT)
    nb = jnp.where(lane[:1, :] < E, nb, 0.0)
    pnb = nb
    s = 1
    while s < LANES:
        pnb = pnb + jnp.concatenate(
            [jnp.zeros((1, s), jnp.float32), pnb[:, :-s]], axis=1)
        s *= 2
    pext = pnb - nb

    r8 = 8.0 * jnp.floor((counts + 7.0) / 8.0)
    r8 = jnp.where(lane[:1, :] < E, r8, 0.0)
    po8 = r8
    s = 1
    while s < LANES:
        po8 = po8 + jnp.concatenate(
            [jnp.zeros((1, s), jnp.float32), po8[:, :-s]], axis=1)
        s *= 2
    off8 = po8 - r8

    slot = excl + off8
    pos0 = jnp.sum(oh0 * slot, axis=1, keepdims=True)
    pos1 = jnp.sum(oh1 * slot, axis=1, keepdims=True)

    nact = jnp.sum(jnp.where(lane[:1, :] == E - 1, pnb, 0.0),
                   axis=1, keepdims=True)
    bf = lane[:1, :].astype(jnp.float32)
    be = jnp.zeros((1, LANES), jnp.float32)
    for e in range(E):
        pnb_e = jnp.sum(jnp.where(lane[:1, :] == e, pnb, 0.0),
                        axis=1, keepdims=True)
        be = be + (bf >= pnb_e).astype(jnp.float32)
    be = jnp.minimum(be, float(E - 1))

    be_prev = jnp.concatenate([be[:, :1] - 1.0, be[:, :NBMAX - 1],
                               jnp.zeros((1, LANES - NBMAX), jnp.float32)],
                              axis=1)
    sta = jnp.where(lane[:1, :] < NBMAX,
                    (be != be_prev).astype(jnp.float32), 0.0)
    rid = sta
    s = 1
    while s < LANES:
        rid = rid + jnp.concatenate(
            [jnp.zeros((1, s), jnp.float32), rid[:, :-s]], axis=1)
        s *= 2
    rid = rid - 1.0
    rpf = jnp.sum(jnp.where(lane[:1, :] == NBMAX - 1, rid + 1.0, 0.0),
                  axis=1, keepdims=True)
    rex = jnp.zeros((1, LANES), jnp.float32)
    for r in range(E):
        rex_r = jnp.min(jnp.where((rid == r) & (lane[:1, :] < NBMAX),
                                  be, float(E)), axis=1, keepdims=True)
        rex = rex + (lane[:1, :] == r).astype(jnp.float32) * rex_r
    rex = jnp.minimum(rex, float(E - 1))

    rowoff = jnp.zeros((1, LANES), jnp.float32)
    for e in range(E):
        off8_e = jnp.sum(jnp.where(lane[:1, :] == e, off8, 0.0),
                         axis=1, keepdims=True)
        pext_e = jnp.sum(jnp.where(lane[:1, :] == e, pext, 0.0),
                         axis=1, keepdims=True)
        rowoff = rowoff + (be == e).astype(jnp.float32) * (
            off8_e + (bf - pext_e) * BT)
    rowoff = jnp.clip(rowoff, 0.0, float(PX - BT))

    meta = jnp.concatenate([be, rex, rid, sta, rowoff,
                            jnp.broadcast_to(nact, (1, LANES)),
                            jnp.broadcast_to(rpf, (1, LANES))], axis=0)

    pos0_ref[...] = pos0.astype(jnp.int32)
    pos1_ref[...] = pos1.astype(jnp.int32)
    w0_ref[...] = jnp.broadcast_to(w0, (T, 16))
    w1_ref[...] = jnp.broadcast_to(w1, (T, 16))
    meta_ref[...] = meta.astype(jnp.int32)


def _routing(x, gw_pad):
    return pl.pallas_call(
        _routing_body,
        out_shape=[
            jax.ShapeDtypeStruct((T, 1), jnp.int32),
            jax.ShapeDtypeStruct((T, 1), jnp.int32),
            jax.ShapeDtypeStruct((T, 16), jnp.float32),
            jax.ShapeDtypeStruct((T, 16), jnp.float32),
            jax.ShapeDtypeStruct((7, LANES), jnp.int32),
        ],
    )(x, gw_pad)


def _dispatch_body(x_hbm, p0_hbm, p1_hbm, out_hbm, idx0_v, idx1_v, rows_v, sem):
    c = lax.axis_index("c")
    s = lax.axis_index("s")
    wid = s * 2 + c
    pltpu.sync_copy(p0_hbm.at[wid], idx0_v)
    pltpu.sync_copy(p1_hbm.at[wid], idx1_v)
    pltpu.sync_copy(x_hbm.at[pl.ds(wid * TPT, TPT)], rows_v)
    cp0 = pltpu.async_copy(rows_v, out_hbm.at[idx0_v], sem)
    cp1 = pltpu.async_copy(rows_v, out_hbm.at[idx1_v], sem)
    cp0.wait()
    cp1.wait()


def _dispatch(x, p0, p1):
    mesh = plsc.VectorSubcoreMesh(core_axis_name="c", subcore_axis_name="s")
    fn = functools.partial(
        pl.kernel,
        out_type=jax.ShapeDtypeStruct((PX, D), jnp.float32),
        mesh=mesh,
        scratch_types=[
            pltpu.VMEM((TPT,), jnp.int32),
            pltpu.VMEM((TPT,), jnp.int32),
            pltpu.VMEM((TPT, D), jnp.float32),
            pltpu.SemaphoreType.DMA,
        ],
    )(_dispatch_body)
    return fn(x, p0, p1)


NSLOT = 2


def _gmm_body(bex_ref, rex_ref, rid_ref, sta_ref, roff_ref, nact_ref, rpf_ref,
              xs_hbm, w1_hbm, w3_hbm, w2_hbm, out_ref,
              acc_ref, xs_v, wb1, wb3, wb2, s1, s3, s2, xsem, osem):
    f = pl.program_id(0)
    j = pl.program_id(1)

    @pl.when((f == 0) & (j == 0))
    def _():
        pltpu.make_async_copy(xs_hbm, xs_v, xsem).start()
    ecur = bex_ref[j]
    r = rid_ref[j]
    rpf = rpf_ref[0]
    g = f * rpf + r
    slot = lax.rem(g, NSLOT)

    def copies(e, fidx, sl):
        return (
            pltpu.make_async_copy(
                w1_hbm.at[e, pl.ds(fidx * FFB, FFB)], wb1.at[sl], s1.at[sl]),
            pltpu.make_async_copy(
                w3_hbm.at[e, pl.ds(fidx * FFB, FFB)], wb3.at[sl], s3.at[sl]),
            pltpu.make_async_copy(
                w2_hbm.at[e, :, pl.ds(fidx * FFB, FFB)], wb2.at[sl], s2.at[sl]),
        )

    def issue_run(gt):
        f_t = lax.div(gt, rpf)
        r_t = gt - f_t * rpf

        @pl.when(f_t < NF)
        def _():
            for cp in copies(rex_ref[r_t], f_t, lax.rem(gt, NSLOT)):
                cp.start()

    @pl.when(sta_ref[j] == 1)
    def _():
        @pl.when((f == 0) & (j == 0))
        def _():
            for k in range(NSLOT - 1):
                issue_run(k)
            pltpu.make_async_copy(xs_hbm, xs_v, xsem).wait()
        for cp in copies(ecur, f, slot):
            cp.wait()
        issue_run(g + NSLOT - 1)

    @pl.when(j < nact_ref[0])
    def _():
        roff = pl.multiple_of(roff_ref[j], 8)
        x = xs_v[pl.ds(roff, BT)]
        w1c = wb1[slot]
        w3c = wb3[slot]
        w2c = wb2[slot]
        h1 = lax.dot_general(x, w1c, (((1,), (1,)), ((), ())),
                             preferred_element_type=jnp.float32)
        h3 = lax.dot_general(x, w3c, (((1,), (1,)), ((), ())),
                             preferred_element_type=jnp.float32)
        h = h1 * lax.logistic(h1) * h3
        y = lax.dot_general(h, w2c, (((1,), (1,)), ((), ())),
                            preferred_element_type=jnp.float32)

        @pl.when(f == 0)
        def _():
            acc_ref[pl.ds(j, 1)] = y[None]

        @pl.when(f > 0)
        def _():
            acc_ref[pl.ds(j, 1)] += y[None]

        @pl.when(f == NF - 1)
        def _():
            @pl.when(j > 0)
            def _():
                roffp = pl.multiple_of(roff_ref[j - 1], 8)
                pltpu.make_async_copy(
                    acc_ref.at[j - 1], out_ref.at[pl.ds(roffp, BT)], osem).wait()
            pltpu.make_async_copy(
                acc_ref.at[j], out_ref.at[pl.ds(roff, BT)], osem).start()

    @pl.when((f == NF - 1) & (j == NBMAX - 1))
    def _():
        jl = nact_ref[0] - 1
        roffl = pl.multiple_of(roff_ref[jl], 8)
        pltpu.make_async_copy(
            acc_ref.at[jl], out_ref.at[pl.ds(roffl, BT)], osem).wait()


def _gmm(bex, rex, rid, sta, roff, nact, rpf, xs, w1, w3, w2):
    grid_spec = pltpu.PrefetchScalarGridSpec(
        num_scalar_prefetch=7,
        grid=(NF, NBMAX),
        in_specs=[
            pl.BlockSpec(memory_space=pl.ANY),
            pl.BlockSpec(memory_space=pl.ANY),
            pl.BlockSpec(memory_space=pl.ANY),
            pl.BlockSpec(memory_space=pl.ANY),
        ],
        out_specs=pl.BlockSpec(memory_space=pl.ANY),
        scratch_shapes=[
            pltpu.VMEM((NBMAX, BT, D), jnp.float32),
            pltpu.VMEM((PX, D), jnp.float32),
            pltpu.VMEM((NSLOT, FFB, D), jnp.float32),
            pltpu.VMEM((NSLOT, FFB, D), jnp.float32),
            pltpu.VMEM((NSLOT, D, FFB), jnp.float32),
            pltpu.SemaphoreType.DMA((NSLOT,)),
            pltpu.SemaphoreType.DMA((NSLOT,)),
            pltpu.SemaphoreType.DMA((NSLOT,)),
            pltpu.SemaphoreType.DMA,
            pltpu.SemaphoreType.DMA,
        ],
    )
    return pl.pallas_call(
        _gmm_body,
        grid_spec=grid_spec,
        out_shape=jax.ShapeDtypeStruct((PX, D), jnp.float32),
        compiler_params=pltpu.CompilerParams(
            dimension_semantics=("arbitrary", "arbitrary")),
    )(bex, rex, rid, sta, roff, nact, rpf, xs, w1, w3, w2)


def _combine_body(ys_hbm, p0_hbm, p1_hbm, w0_hbm, w1_hbm, out_hbm,
                  idx0_v, idx1_v, w0_v, w1_v, g0a, g1a, g0b, g1b, sem0, sem1):
    c = lax.axis_index("c")
    s = lax.axis_index("s")
    wid = s * 2 + c
    pltpu.sync_copy(p0_hbm.at[wid], idx0_v)
    pltpu.sync_copy(p1_hbm.at[wid], idx1_v)
    pltpu.sync_copy(w0_hbm.at[wid], w0_v)
    pltpu.sync_copy(w1_hbm.at[wid], w1_v)
    qn = TPT // 4
    bufs = ((g0a, g1a), (g0b, g1b))

    def gather(q):
        b0, b1 = bufs[q % 2]
        return (
            pltpu.async_copy(ys_hbm.at[idx0_v.at[pl.ds(q * qn, qn)]], b0, sem0),
            pltpu.async_copy(ys_hbm.at[idx1_v.at[pl.ds(q * qn, qn)]], b1, sem1),
        )

    pend = {0: gather(0)}
    for q in range(4):
        if q + 1 < 4:
            pend[q + 1] = gather(q + 1)
        for cp in pend[q]:
            cp.wait()
        b0, b1 = bufs[q % 2]

        def row_body(rr, _, q=q, b0=b0, b1=b1):
            for r in (2 * rr, 2 * rr + 1):
                a = w0_v[q * qn + r, :]
                bw = w1_v[q * qn + r, :]
                for cc in range(D // 16):
                    sl = pl.ds(cc * 16, 16)
                    b0[r, sl] = a * b0[r, sl] + bw * b1[r, sl]
            return 0

        lax.fori_loop(0, qn // 2, row_body, 0)
        pltpu.sync_copy(b0, out_hbm.at[pl.ds(wid * TPT + q * qn, qn)])


def _combine(ys, p0, p1, w0m, w1m):
    mesh = plsc.VectorSubcoreMesh(core_axis_name="c", subcore_axis_name="s")
    qn = TPT // 4
    fn = functools.partial(
        pl.kernel,
        out_type=jax.ShapeDtypeStruct((T, D), jnp.float32),
        mesh=mesh,
        scratch_types=[
            pltpu.VMEM((TPT,), jnp.int32),
            pltpu.VMEM((TPT,), jnp.int32),
            pltpu.VMEM((TPT, 16), jnp.float32),
            pltpu.VMEM((TPT, 16), jnp.float32),
            pltpu.VMEM((qn, D), jnp.float32),
            pltpu.VMEM((qn, D), jnp.float32),
            pltpu.VMEM((qn, D), jnp.float32),
            pltpu.VMEM((qn, D), jnp.float32),
            pltpu.SemaphoreType.DMA,
            pltpu.SemaphoreType.DMA,
        ],
    )(_combine_body)
    return fn(ys, p0, p1, w0m, w1m)


def kernel(hidden_states, gate_w, w1, w2, w3):
    b, s, d = hidden_states.shape
    x = hidden_states.reshape(-1, d)
    gw_pad = jnp.pad(gate_w, ((0, LANES - E), (0, 0)))
    pos0, pos1, w0c, w1c, meta = _routing(x, gw_pad)
    p0 = pos0.reshape(NTILES, TPT)
    p1 = pos1.reshape(NTILES, TPT)
    w0m = w0c.reshape(NTILES, TPT, 16)
    w1m = w1c.reshape(NTILES, TPT, 16)
    bex = meta[0, :NBMAX]
    rex = meta[1, :E]
    rid = meta[2, :NBMAX]
    sta = meta[3, :NBMAX]
    roff = meta[4, :NBMAX]
    nact = meta[5, :1]
    rpf = meta[6, :1]
    xs = _dispatch(x, p0, p1)
    ys = _gmm(bex, rex, rid, sta, roff, nact, rpf, xs, w1, w3, w2)
    out = _combine(ys, p0, p1, w0m, w1m)
    return out.reshape(b, s, d)
